# Initial kernel scaffold; baseline (speedup 1.0000x reference)
import jax
import jax.numpy as jnp
from jax.experimental import pallas as pl


def kernel(x, edge_index, batch, W1, a1s, a1d, b1, W2, a2s, a2d, b2, W3, a3s, a3d, b3, Wn, bn, W0, b0, W4, b4):
    raise NotImplementedError("write your pallas kernel here")


# speedup vs baseline: 12.0537x; 12.0537x over previous
import dataclasses

import jax
import jax.numpy as jnp
from jax import lax
from jax.experimental import pallas as pl
from jax.experimental.pallas import tpu as pltpu
from jax.experimental.pallas import tpu_sc as plsc

N = 10000
NPAD = 10240
E = 170000
CHA = 42
EPC = 32 * CHA * 128
HID = 256
NG = 64
R = 1024
NBLK = NPAD // R

_HIGH = lax.Precision.HIGHEST


def _mm(x, w, dims):
    return lax.dot_general(x, w, (dims, ((), ())), precision=_HIGH,
                           preferred_element_type=jnp.float32)


def _sc_params():
    cp = pltpu.CompilerParams()
    if "needs_layout_passes" in pltpu.CompilerParams.__dataclass_fields__:
        cp = dataclasses.replace(cp, needs_layout_passes=False)
    return cp


def _mm_first_body(x_ref, w_ref, as_ref, ad_ref,
                   h2_ref, asv_ref, adv_ref, mas_ref, mad_ref):
    i = pl.program_id(0)
    h = _mm(x_ref[...], w_ref[...], ((1,), (1,)))
    h2_ref[0] = h[:, :128]
    h2_ref[1] = h[:, 128:]
    asv = _mm(h, as_ref[...], ((1,), (0,)))
    adv = _mm(h, ad_ref[...], ((1,), (0,)))
    asv_ref[...] = asv
    adv_ref[...] = adv

    @pl.when(i == 0)
    def _():
        mas_ref[...] = jnp.full((16,), -3e38, jnp.float32)
        mad_ref[...] = jnp.full((16,), -3e38, jnp.float32)

    mas_ref[...] = jnp.maximum(mas_ref[...], jnp.max(asv))
    mad_ref[...] = jnp.maximum(mad_ref[...], jnp.max(adv))


def _mm_first(x_pad, w, a_s, a_d):
    return pl.pallas_call(
        _mm_first_body,
        grid=(NBLK,),
        in_specs=[
            pl.BlockSpec((R, HID), lambda i: (i, 0)),
            pl.BlockSpec((HID, HID), lambda i: (0, 0)),
            pl.BlockSpec((HID,), lambda i: (0,)),
            pl.BlockSpec((HID,), lambda i: (0,)),
        ],
        out_specs=[
            pl.BlockSpec((2, R, 128), lambda i: (0, i, 0)),
            pl.BlockSpec((R,), lambda i: (i,)),
            pl.BlockSpec((R,), lambda i: (i,)),
            pl.BlockSpec((16,), lambda i: (0,)),
            pl.BlockSpec((16,), lambda i: (0,)),
        ],
        out_shape=[
            jax.ShapeDtypeStruct((2, NPAD, 128), jnp.float32),
            jax.ShapeDtypeStruct((NPAD,), jnp.float32),
            jax.ShapeDtypeStruct((NPAD,), jnp.float32),
            jax.ShapeDtypeStruct((16,), jnp.float32),
            jax.ShapeDtypeStruct((16,), jnp.float32),
        ],
    )(x_pad, w, a_s, a_d)


def _mm_mid_body(acc_ref, s_ref, b_ref, w_ref, as_ref, ad_ref,
                 h2_ref, asv_ref, adv_ref, mas_ref, mad_ref):
    i = pl.program_id(0)
    s = s_ref[0] + s_ref[1]
    rcp = 1.0 / jnp.maximum(s, 1e-30)
    b = b_ref[...]
    x0 = jnp.maximum(acc_ref[0] * rcp[:, None] + b[None, :128], 0.0)
    x1 = jnp.maximum(acc_ref[1] * rcp[:, None] + b[None, 128:], 0.0)
    w = w_ref[...]
    h = _mm(x0, w[:, :128], ((1,), (1,))) + _mm(x1, w[:, 128:], ((1,), (1,)))
    h2_ref[0] = h[:, :128]
    h2_ref[1] = h[:, 128:]
    asv = _mm(h, as_ref[...], ((1,), (0,)))
    adv = _mm(h, ad_ref[...], ((1,), (0,)))
    asv_ref[...] = asv
    adv_ref[...] = adv

    @pl.when(i == 0)
    def _():
        mas_ref[...] = jnp.full((16,), -3e38, jnp.float32)
        mad_ref[...] = jnp.full((16,), -3e38, jnp.float32)

    mas_ref[...] = jnp.maximum(mas_ref[...], jnp.max(asv))
    mad_ref[...] = jnp.maximum(mad_ref[...], jnp.max(adv))


def _mm_mid(acc, s2, b, w, a_s, a_d):
    return pl.pallas_call(
        _mm_mid_body,
        grid=(NBLK,),
        in_specs=[
            pl.BlockSpec((2, R, 128), lambda i: (0, i, 0)),
            pl.BlockSpec((2, R), lambda i: (0, i)),
            pl.BlockSpec((HID,), lambda i: (0,)),
            pl.BlockSpec((HID, HID), lambda i: (0, 0)),
            pl.BlockSpec((HID,), lambda i: (0,)),
            pl.BlockSpec((HID,), lambda i: (0,)),
        ],
        out_specs=[
            pl.BlockSpec((2, R, 128), lambda i: (0, i, 0)),
            pl.BlockSpec((R,), lambda i: (i,)),
            pl.BlockSpec((R,), lambda i: (i,)),
            pl.BlockSpec((16,), lambda i: (0,)),
            pl.BlockSpec((16,), lambda i: (0,)),
        ],
        out_shape=[
            jax.ShapeDtypeStruct((2, NPAD, 128), jnp.float32),
            jax.ShapeDtypeStruct((NPAD,), jnp.float32),
            jax.ShapeDtypeStruct((NPAD,), jnp.float32),
            jax.ShapeDtypeStruct((16,), jnp.float32),
            jax.ShapeDtypeStruct((16,), jnp.float32),
        ],
    )(acc, s2, b, w, a_s, a_d)


def _sc_soft_body(srcs_hbm, dsts_hbm, asrc_hbm, adst_hbm, mas_hbm, mad_hbm,
                  ex_hbm, sout_hbm,
                  asrc_t, adst_t, src_t, dst_t, ex_t, mas_t, mad_t, z_t, s_sh):
    c = lax.axis_index("c")
    sid = lax.axis_index("s")
    w = 2 * sid + c

    pltpu.sync_copy(srcs_hbm.at[w], src_t)
    pltpu.sync_copy(dsts_hbm.at[w], dst_t)
    pltpu.sync_copy(asrc_hbm, asrc_t)
    pltpu.sync_copy(adst_hbm, adst_t)
    pltpu.sync_copy(mas_hbm, mas_t)
    pltpu.sync_copy(mad_hbm, mad_t)

    zv = jnp.zeros((16,), jnp.float32)

    @pl.loop(0, 640, step=16)
    def _(i):
        z_t[pl.ds(i, 16)] = zv

    pltpu.sync_copy(z_t, s_sh.at[pl.ds(sid * 640, 640)])
    plsc.subcore_barrier()

    t_m = mas_t[...] + mad_t[...]
    m_vec = jnp.maximum(t_m, 0.2 * t_m)

    @pl.loop(0, CHA)
    def _(j):
        for k in range(8):
            s16 = src_t[j, pl.ds(k * 16, 16)]
            d16 = dst_t[j, pl.ds(k * 16, 16)]
            av = plsc.load_gather(asrc_t, [s16])
            bv = plsc.load_gather(adst_t, [d16])
            t = av + bv
            e = jnp.maximum(t, 0.2 * t)
            ex_t[j, pl.ds(k * 16, 16)] = jnp.exp(e - m_vec)
        pltpu.sync_copy(ex_t.at[j], s_sh.at[dst_t.at[j]], add=True)

    pltpu.sync_copy(ex_t, ex_hbm.at[w])
    plsc.subcore_barrier()
    pltpu.sync_copy(s_sh.at[pl.ds(sid * 640, 640)],
                    sout_hbm.at[c].at[pl.ds(sid * 640, 640)])


def _sc_soft(srcs3d, dsts3d, asv, adv, mas, mad):
    mesh = plsc.VectorSubcoreMesh(core_axis_name="c", subcore_axis_name="s")
    f = pl.kernel(
        _sc_soft_body,
        compiler_params=_sc_params(),
        out_type=[
            jax.ShapeDtypeStruct((32, CHA, 128), jnp.float32),
            jax.ShapeDtypeStruct((2, NPAD), jnp.float32),
        ],
        mesh=mesh,
        scratch_types=[
            pltpu.VMEM((NPAD,), jnp.float32),
            pltpu.VMEM((NPAD,), jnp.float32),
            pltpu.VMEM((CHA, 128), jnp.int32),
            pltpu.VMEM((CHA, 128), jnp.int32),
            pltpu.VMEM((CHA, 128), jnp.float32),
            pltpu.VMEM((16,), jnp.float32),
            pltpu.VMEM((16,), jnp.float32),
            pltpu.VMEM((640,), jnp.float32),
            pltpu.VMEM_SHARED((NPAD,), jnp.float32),
        ],
    )
    return f(srcs3d, dsts3d, asv, adv, mas, mad)


def _sc_agg_body(h2_hbm, srcs_hbm, dsts_hbm, ex_hbm, acc_hbm,
                 src_t, dst_t, exj_t, rows_t, sem, acc_sh):
    c = lax.axis_index("c")
    sid = lax.axis_index("s")

    pltpu.sync_copy(srcs_hbm.at[sid], src_t)
    pltpu.sync_copy(dsts_hbm.at[sid], dst_t)
    exv = ex_hbm.at[sid]

    zv = jnp.zeros((16,), jnp.float32)

    @pl.loop(0, 128)
    def _(r):
        for k in range(8):
            rows_t[r, pl.ds(k * 16, 16)] = zv

    for k in range(5):
        pltpu.sync_copy(rows_t, acc_sh.at[pl.ds(sid * 640 + k * 128, 128)])
    plsc.subcore_barrier()

    hsel = h2_hbm.at[c]

    @pl.loop(0, 2 * CHA)
    def _(j):
        d1 = pltpu.async_copy(hsel.at[src_t.at[j]], rows_t, sem)
        d2 = pltpu.async_copy(exv.at[pl.ds(j * 128, 128)], exj_t, sem)
        d1.wait()
        d2.wait()

        @pl.loop(0, 128)
        def _(r):
            av = plsc.load_gather(exj_t, [jnp.full((16,), r, jnp.int32)])
            for k in range(8):
                rows_t[r, pl.ds(k * 16, 16)] = (
                    rows_t[r, pl.ds(k * 16, 16)] * av)

        pltpu.sync_copy(rows_t, acc_sh.at[dst_t.at[j]], add=True)

    plsc.subcore_barrier()
    pltpu.sync_copy(acc_sh.at[pl.ds(sid * 640, 640)],
                    acc_hbm.at[c].at[pl.ds(sid * 640, 640)])


def _sc_agg(h2, srcs3db, dsts3db, exb):
    mesh = plsc.VectorSubcoreMesh(core_axis_name="c", subcore_axis_name="s")
    f = pl.kernel(
        _sc_agg_body,
        compiler_params=_sc_params(),
        out_type=jax.ShapeDtypeStruct((2, NPAD, 128), jnp.float32),
        mesh=mesh,
        scratch_types=[
            pltpu.VMEM((2 * CHA, 128), jnp.int32),
            pltpu.VMEM((2 * CHA, 128), jnp.int32),
            pltpu.VMEM((128,), jnp.float32),
            pltpu.VMEM((128, 128), jnp.float32),
            pltpu.SemaphoreType.DMA,
            pltpu.VMEM_SHARED((NPAD, 128), jnp.float32),
        ],
    )
    return f(h2, srcs3db, dsts3db, exb)


def _gat_edges(h2, srcs3d, dsts3d, srcs3db, dsts3db, asv, adv, mas, mad):
    ex, s2 = _sc_soft(srcs3d, dsts3d, asv, adv, mas, mad)
    acc = _sc_agg(h2, srcs3db, dsts3db, ex.reshape(16, 2 * CHA * 128))
    return acc, s2


def _final_body(batch_sm, acc_ref, s_ref, b3_ref, bv_ref, x_ref,
                w0_ref, b0_ref, wn_ref, bn_ref, w4_ref, b4_ref,
                out_ref, hg_scr, news_scr):
    i = pl.program_id(0)

    @pl.when(i == 0)
    def _():
        hg_scr[...] = jnp.zeros((NG, HID), jnp.float32)

    s = s_ref[0] + s_ref[1]
    rcp = 1.0 / jnp.maximum(s, 1e-30)
    b3 = b3_ref[...]
    h0 = jnp.maximum(acc_ref[0] * rcp[:, None] + b3[None, :128], 0.0)
    h1 = jnp.maximum(acc_ref[1] * rcp[:, None] + b3[None, 128:], 0.0)
    h3 = jnp.concatenate([h0, h1], axis=1)
    bv = bv_ref[...]

    g_lo = batch_sm[i * R]
    g_hi = jnp.minimum(batch_sm[i * R + R - 1], NG - 1)

    def seg_body(g, _):
        mask = bv == g
        mx = jnp.max(jnp.where(mask, h3, 0.0), axis=0, keepdims=True)
        hg_scr[pl.ds(g, 1), :] = jnp.maximum(hg_scr[pl.ds(g, 1), :], mx)
        return 0

    lax.fori_loop(g_lo, g_hi + 1, seg_body, 0)

    @pl.when(i == NBLK - 1)
    def _():
        def root_body(g, _):
            def bs(_, lohi):
                lo, hi = lohi
                mid = (lo + hi) // 2
                p = batch_sm[mid] < g
                return jnp.where(p, mid + 1, lo), jnp.where(p, hi, mid)

            lo, _hi = lax.fori_loop(0, 14, bs, (0, NPAD))
            news_scr[pl.ds(g, 1), :] = x_ref[pl.ds(lo, 1), :]
            return 0

        lax.fori_loop(0, NG, root_body, 0)

        hgf = jnp.maximum(
            _mm(hg_scr[...], w0_ref[...], ((1,), (1,))) + b0_ref[...][None, :],
            0.0)
        newsh = jnp.maximum(
            _mm(news_scr[...], wn_ref[...], ((1,), (1,))) + bn_ref[...][None, :],
            0.0)
        w4 = w4_ref[...]
        logit = (_mm(hgf, w4[:, :HID], ((1,), (1,))) +
                 _mm(newsh, w4[:, HID:], ((1,), (1,))) + b4_ref[...][None, :])
        out_ref[...] = jax.nn.sigmoid(logit)


def _final(batch_pad, acc, s2, b3, x_pad, w0, b0, wn, bn, w4, b4):
    grid_spec = pltpu.PrefetchScalarGridSpec(
        num_scalar_prefetch=1,
        grid=(NBLK,),
        in_specs=[
            pl.BlockSpec((2, R, 128), lambda i, b: (0, i, 0)),
            pl.BlockSpec((2, R), lambda i, b: (0, i)),
            pl.BlockSpec((HID,), lambda i, b: (0,)),
            pl.BlockSpec((R, 1), lambda i, b: (i, 0)),
            pl.BlockSpec((NPAD, HID), lambda i, b: (0, 0)),
            pl.BlockSpec((HID, HID), lambda i, b: (0, 0)),
            pl.BlockSpec((HID,), lambda i, b: (0,)),
            pl.BlockSpec((HID, HID), lambda i, b: (0, 0)),
            pl.BlockSpec((HID,), lambda i, b: (0,)),
            pl.BlockSpec((1, 2 * HID), lambda i, b: (0, 0)),
            pl.BlockSpec((1,), lambda i, b: (0,)),
        ],
        out_specs=pl.BlockSpec((NG, 1), lambda i, b: (0, 0)),
        scratch_shapes=[
            pltpu.VMEM((NG, HID), jnp.float32),
            pltpu.VMEM((NG, HID), jnp.float32),
        ],
    )
    return pl.pallas_call(
        _final_body,
        grid_spec=grid_spec,
        out_shape=jax.ShapeDtypeStruct((NG, 1), jnp.float32),
    )(batch_pad, acc, s2, b3, batch_pad[:, None], x_pad,
      w0, b0, wn, bn, w4, b4)


@jax.jit
def _run(x, edge_index, batch, W1, a1s, a1d, b1, W2, a2s, a2d, b2,
         W3, a3s, a3d, b3, Wn, bn, W0, b0, W4, b4):
    loops = jnp.arange(N, dtype=jnp.int32)
    ei = edge_index.astype(jnp.int32)
    src = jnp.concatenate([ei[0], loops])
    dst = jnp.concatenate([ei[1], loops])
    npad_e = EPC - E
    src = jnp.concatenate([src, jnp.zeros((npad_e,), jnp.int32)])
    dst = jnp.concatenate([dst, jnp.full((npad_e,), N, jnp.int32)])
    srcs3d = src.reshape(32, CHA, 128)
    dsts3d = dst.reshape(32, CHA, 128)
    srcs3db = src.reshape(16, 2 * CHA, 128)
    dsts3db = dst.reshape(16, 2 * CHA, 128)

    x_pad = jnp.zeros((NPAD, HID), jnp.float32).at[:N].set(x)
    batch_pad = jnp.concatenate(
        [batch.astype(jnp.int32), jnp.full((NPAD - N,), NG, jnp.int32)])

    h2, asv, adv, mas, mad = _mm_first(x_pad, W1, a1s, a1d)
    acc, s2 = _gat_edges(h2, srcs3d, dsts3d, srcs3db, dsts3db,
                         asv, adv, mas, mad)
    h2, asv, adv, mas, mad = _mm_mid(acc, s2, b1, W2, a2s, a2d)
    acc, s2 = _gat_edges(h2, srcs3d, dsts3d, srcs3db, dsts3db,
                         asv, adv, mas, mad)
    h2, asv, adv, mas, mad = _mm_mid(acc, s2, b2, W3, a3s, a3d)
    acc, s2 = _gat_edges(h2, srcs3d, dsts3d, srcs3db, dsts3db,
                         asv, adv, mas, mad)
    return _final(batch_pad, acc, s2, b3, x_pad, W0, b0, Wn, bn, W4, b4)


def kernel(x, edge_index, batch, W1, a1s, a1d, b1, W2, a2s, a2d, b2,
           W3, a3s, a3d, b3, Wn, bn, W0, b0, W4, b4):
    return _run(x, edge_index, batch, W1, a1s, a1d, b1, W2, a2s, a2d, b2,
                W3, a3s, a3d, b3, Wn, bn, W0, b0, W4, b4)


# speedup vs baseline: 15.4261x; 1.2798x over previous
import dataclasses

import jax
import jax.numpy as jnp
from jax import lax
from jax.experimental import pallas as pl
from jax.experimental.pallas import tpu as pltpu
from jax.experimental.pallas import tpu_sc as plsc

N = 10000
NPAD = 10240
E = 170000
CHA = 42
EPC = 32 * CHA * 128
HID = 256
NG = 64
R = 1024
NBLK = NPAD // R

_HIGH = lax.Precision.HIGHEST


def _mm(x, w, dims):
    return lax.dot_general(x, w, (dims, ((), ())), precision=_HIGH,
                           preferred_element_type=jnp.float32)


def _sc_params():
    cp = pltpu.CompilerParams()
    if "needs_layout_passes" in pltpu.CompilerParams.__dataclass_fields__:
        cp = dataclasses.replace(cp, needs_layout_passes=False)
    return cp


def _mm_first_body(x_ref, w_ref, as_ref, ad_ref,
                   h2_ref, asv_ref, adv_ref, mas_ref, mad_ref):
    i = pl.program_id(0)
    h = _mm(x_ref[...], w_ref[...], ((1,), (1,)))
    h2_ref[0] = h[:, :128]
    h2_ref[1] = h[:, 128:]
    asv = _mm(h, as_ref[...], ((1,), (0,)))
    adv = _mm(h, ad_ref[...], ((1,), (0,)))
    asv_ref[...] = asv
    adv_ref[...] = adv

    @pl.when(i == 0)
    def _():
        mas_ref[...] = jnp.full((16,), -3e38, jnp.float32)
        mad_ref[...] = jnp.full((16,), -3e38, jnp.float32)

    mas_ref[...] = jnp.maximum(mas_ref[...], jnp.max(asv))
    mad_ref[...] = jnp.maximum(mad_ref[...], jnp.max(adv))


def _mm_first(x_pad, w, a_s, a_d):
    return pl.pallas_call(
        _mm_first_body,
        grid=(NBLK,),
        in_specs=[
            pl.BlockSpec((R, HID), lambda i: (i, 0)),
            pl.BlockSpec((HID, HID), lambda i: (0, 0)),
            pl.BlockSpec((HID,), lambda i: (0,)),
            pl.BlockSpec((HID,), lambda i: (0,)),
        ],
        out_specs=[
            pl.BlockSpec((2, R, 128), lambda i: (0, i, 0)),
            pl.BlockSpec((R,), lambda i: (i,)),
            pl.BlockSpec((R,), lambda i: (i,)),
            pl.BlockSpec((16,), lambda i: (0,)),
            pl.BlockSpec((16,), lambda i: (0,)),
        ],
        out_shape=[
            jax.ShapeDtypeStruct((2, NPAD, 128), jnp.float32),
            jax.ShapeDtypeStruct((NPAD,), jnp.float32),
            jax.ShapeDtypeStruct((NPAD,), jnp.float32),
            jax.ShapeDtypeStruct((16,), jnp.float32),
            jax.ShapeDtypeStruct((16,), jnp.float32),
        ],
    )(x_pad, w, a_s, a_d)


def _mm_mid_body(acc_ref, s_ref, b_ref, w_ref, as_ref, ad_ref,
                 h2_ref, asv_ref, adv_ref, mas_ref, mad_ref):
    i = pl.program_id(0)
    s = s_ref[0] + s_ref[1]
    rcp = 1.0 / jnp.maximum(s, 1e-30)
    b = b_ref[...]
    x0 = jnp.maximum(acc_ref[0] * rcp[:, None] + b[None, :128], 0.0)
    x1 = jnp.maximum(acc_ref[1] * rcp[:, None] + b[None, 128:], 0.0)
    w = w_ref[...]
    h = _mm(x0, w[:, :128], ((1,), (1,))) + _mm(x1, w[:, 128:], ((1,), (1,)))
    h2_ref[0] = h[:, :128]
    h2_ref[1] = h[:, 128:]
    asv = _mm(h, as_ref[...], ((1,), (0,)))
    adv = _mm(h, ad_ref[...], ((1,), (0,)))
    asv_ref[...] = asv
    adv_ref[...] = adv

    @pl.when(i == 0)
    def _():
        mas_ref[...] = jnp.full((16,), -3e38, jnp.float32)
        mad_ref[...] = jnp.full((16,), -3e38, jnp.float32)

    mas_ref[...] = jnp.maximum(mas_ref[...], jnp.max(asv))
    mad_ref[...] = jnp.maximum(mad_ref[...], jnp.max(adv))


def _mm_mid(acc, s2, b, w, a_s, a_d):
    return pl.pallas_call(
        _mm_mid_body,
        grid=(NBLK,),
        in_specs=[
            pl.BlockSpec((2, R, 128), lambda i: (0, i, 0)),
            pl.BlockSpec((2, R), lambda i: (0, i)),
            pl.BlockSpec((HID,), lambda i: (0,)),
            pl.BlockSpec((HID, HID), lambda i: (0, 0)),
            pl.BlockSpec((HID,), lambda i: (0,)),
            pl.BlockSpec((HID,), lambda i: (0,)),
        ],
        out_specs=[
            pl.BlockSpec((2, R, 128), lambda i: (0, i, 0)),
            pl.BlockSpec((R,), lambda i: (i,)),
            pl.BlockSpec((R,), lambda i: (i,)),
            pl.BlockSpec((16,), lambda i: (0,)),
            pl.BlockSpec((16,), lambda i: (0,)),
        ],
        out_shape=[
            jax.ShapeDtypeStruct((2, NPAD, 128), jnp.float32),
            jax.ShapeDtypeStruct((NPAD,), jnp.float32),
            jax.ShapeDtypeStruct((NPAD,), jnp.float32),
            jax.ShapeDtypeStruct((16,), jnp.float32),
            jax.ShapeDtypeStruct((16,), jnp.float32),
        ],
    )(acc, s2, b, w, a_s, a_d)


def _sc_soft_body(srcs_hbm, dsts_hbm, asrc_hbm, adst_hbm, mas_hbm, mad_hbm,
                  ex_hbm, sout_hbm,
                  asrc_t, adst_t, src_t, dst_t, ex_t, mas_t, mad_t, z_t, s_sh):
    c = lax.axis_index("c")
    sid = lax.axis_index("s")
    w = 2 * sid + c

    pltpu.sync_copy(srcs_hbm.at[w], src_t)
    pltpu.sync_copy(dsts_hbm.at[w], dst_t)
    pltpu.sync_copy(asrc_hbm, asrc_t)
    pltpu.sync_copy(adst_hbm, adst_t)
    pltpu.sync_copy(mas_hbm, mas_t)
    pltpu.sync_copy(mad_hbm, mad_t)

    zv = jnp.zeros((16,), jnp.float32)

    @pl.loop(0, 640, step=16)
    def _(i):
        z_t[pl.ds(i, 16)] = zv

    pltpu.sync_copy(z_t, s_sh.at[pl.ds(sid * 640, 640)])
    plsc.subcore_barrier()

    t_m = mas_t[...] + mad_t[...]
    m_vec = jnp.maximum(t_m, 0.2 * t_m)

    @pl.loop(0, CHA)
    def _(j):
        for k in range(8):
            s16 = src_t[j, pl.ds(k * 16, 16)]
            d16 = dst_t[j, pl.ds(k * 16, 16)]
            av = plsc.load_gather(asrc_t, [s16])
            bv = plsc.load_gather(adst_t, [d16])
            t = av + bv
            e = jnp.maximum(t, 0.2 * t)
            ex_t[j, pl.ds(k * 16, 16)] = jnp.exp(e - m_vec)
        pltpu.sync_copy(ex_t.at[j], s_sh.at[dst_t.at[j]], add=True)

    pltpu.sync_copy(ex_t, ex_hbm.at[w])
    plsc.subcore_barrier()
    pltpu.sync_copy(s_sh.at[pl.ds(sid * 640, 640)],
                    sout_hbm.at[c].at[pl.ds(sid * 640, 640)])


def _sc_soft(srcs3d, dsts3d, asv, adv, mas, mad):
    mesh = plsc.VectorSubcoreMesh(core_axis_name="c", subcore_axis_name="s")
    f = pl.kernel(
        _sc_soft_body,
        compiler_params=_sc_params(),
        out_type=[
            jax.ShapeDtypeStruct((32, CHA, 128), jnp.float32),
            jax.ShapeDtypeStruct((2, NPAD), jnp.float32),
        ],
        mesh=mesh,
        scratch_types=[
            pltpu.VMEM((NPAD,), jnp.float32),
            pltpu.VMEM((NPAD,), jnp.float32),
            pltpu.VMEM((CHA, 128), jnp.int32),
            pltpu.VMEM((CHA, 128), jnp.int32),
            pltpu.VMEM((CHA, 128), jnp.float32),
            pltpu.VMEM((16,), jnp.float32),
            pltpu.VMEM((16,), jnp.float32),
            pltpu.VMEM((640,), jnp.float32),
            pltpu.VMEM_SHARED((NPAD,), jnp.float32),
        ],
    )
    return f(srcs3d, dsts3d, asv, adv, mas, mad)


def _sc_agg_body(h2_hbm, srcs_hbm, dsts_hbm, ex_hbm, acc_hbm,
                 dst_t, s0_t, s1_t, ex0_t, ex1_t, rows0_t, rows1_t,
                 semg0, semg1, seme0, seme1, semi0, semi1, acc_sh):
    s_t = (s0_t, s1_t)
    exj_t = (ex0_t, ex1_t)
    rows_t = (rows0_t, rows1_t)
    semg = (semg0, semg1)
    seme = (seme0, seme1)
    semi = (semi0, semi1)
    c = lax.axis_index("c")
    sid = lax.axis_index("s")

    pltpu.sync_copy(dsts_hbm.at[sid], dst_t)
    srcv = srcs_hbm.at[sid]
    exv = ex_hbm.at[sid]
    hsel = h2_hbm.at[c]
    last = 2 * CHA - 1

    zv = jnp.zeros((16,), jnp.float32)

    @pl.loop(0, 128)
    def _(r):
        for k in range(8):
            rows_t[0][r, pl.ds(k * 16, 16)] = zv

    for k in range(5):
        pltpu.sync_copy(rows_t[0], acc_sh.at[pl.ds(sid * 640 + k * 128, 128)])
    plsc.subcore_barrier()

    for p in range(2):
        pltpu.sync_copy(srcv.at[pl.ds(p * 128, 128)], s_t[p])
        pltpu.async_copy(hsel.at[s_t[p]], rows_t[p], semg[p])
        pltpu.async_copy(exv.at[pl.ds(p * 128, 128)], exj_t[p], seme[p])

    def scale_scatter(j, p):
        pltpu.make_async_copy(hsel.at[s_t[p]], rows_t[p], semg[p]).wait()
        nj = jnp.minimum(j + 2, last)
        pltpu.async_copy(srcv.at[pl.ds(nj * 128, 128)], s_t[p], semi[p])
        pltpu.make_async_copy(exv.at[pl.ds(0, 128)], exj_t[p], seme[p]).wait()

        @pl.loop(0, 128)
        def _(r):
            av = plsc.load_gather(exj_t[p], [jnp.full((16,), r, jnp.int32)])
            for k in range(8):
                rows_t[p][r, pl.ds(k * 16, 16)] = (
                    rows_t[p][r, pl.ds(k * 16, 16)] * av)

        pltpu.async_copy(exv.at[pl.ds(nj * 128, 128)], exj_t[p], seme[p])
        pltpu.sync_copy(rows_t[p], acc_sh.at[dst_t.at[j]], add=True)
        pltpu.make_async_copy(srcv.at[pl.ds(0, 128)], s_t[p], semi[p]).wait()
        pltpu.async_copy(hsel.at[s_t[p]], rows_t[p], semg[p])

    @pl.loop(0, CHA)
    def _(j2):
        scale_scatter(2 * j2, 0)
        scale_scatter(2 * j2 + 1, 1)

    for p in range(2):
        pltpu.make_async_copy(hsel.at[s_t[p]], rows_t[p], semg[p]).wait()
        pltpu.make_async_copy(exv.at[pl.ds(0, 128)], exj_t[p], seme[p]).wait()

    plsc.subcore_barrier()
    pltpu.sync_copy(acc_sh.at[pl.ds(sid * 640, 640)],
                    acc_hbm.at[c].at[pl.ds(sid * 640, 640)])


def _sc_agg(h2, srcs3db, dsts3db, exb):
    mesh = plsc.VectorSubcoreMesh(core_axis_name="c", subcore_axis_name="s")
    f = pl.kernel(
        _sc_agg_body,
        compiler_params=_sc_params(),
        out_type=jax.ShapeDtypeStruct((2, NPAD, 128), jnp.float32),
        mesh=mesh,
        scratch_types=[
            pltpu.VMEM((2 * CHA, 128), jnp.int32),
            pltpu.VMEM((128,), jnp.int32),
            pltpu.VMEM((128,), jnp.int32),
            pltpu.VMEM((128,), jnp.float32),
            pltpu.VMEM((128,), jnp.float32),
            pltpu.VMEM((128, 128), jnp.float32),
            pltpu.VMEM((128, 128), jnp.float32),
            pltpu.SemaphoreType.DMA,
            pltpu.SemaphoreType.DMA,
            pltpu.SemaphoreType.DMA,
            pltpu.SemaphoreType.DMA,
            pltpu.SemaphoreType.DMA,
            pltpu.SemaphoreType.DMA,
            pltpu.VMEM_SHARED((NPAD, 128), jnp.float32),
        ],
    )
    return f(h2, srcs3db, dsts3db, exb)


def _gat_edges(h2, srcs3d, dsts3d, srcsflat, dsts3db, asv, adv, mas, mad):
    ex, s2 = _sc_soft(srcs3d, dsts3d, asv, adv, mas, mad)
    acc = _sc_agg(h2, srcsflat, dsts3db, ex.reshape(16, 2 * CHA * 128))
    return acc, s2


def _final_body(batch_sm, acc_ref, s_ref, b3_ref, bv_ref, x_ref,
                w0_ref, b0_ref, wn_ref, bn_ref, w4_ref, b4_ref,
                out_ref, hg_scr, news_scr):
    i = pl.program_id(0)

    @pl.when(i == 0)
    def _():
        hg_scr[...] = jnp.zeros((NG, HID), jnp.float32)

    s = s_ref[0] + s_ref[1]
    rcp = 1.0 / jnp.maximum(s, 1e-30)
    b3 = b3_ref[...]
    h0 = jnp.maximum(acc_ref[0] * rcp[:, None] + b3[None, :128], 0.0)
    h1 = jnp.maximum(acc_ref[1] * rcp[:, None] + b3[None, 128:], 0.0)
    h3 = jnp.concatenate([h0, h1], axis=1)
    bv = bv_ref[...]

    g_lo = batch_sm[i * R]
    g_hi = jnp.minimum(batch_sm[i * R + R - 1], NG - 1)

    def seg_body(g, _):
        mask = bv == g
        mx = jnp.max(jnp.where(mask, h3, 0.0), axis=0, keepdims=True)
        hg_scr[pl.ds(g, 1), :] = jnp.maximum(hg_scr[pl.ds(g, 1), :], mx)
        return 0

    lax.fori_loop(g_lo, g_hi + 1, seg_body, 0)

    @pl.when(i == NBLK - 1)
    def _():
        def root_body(g, _):
            def bs(_, lohi):
                lo, hi = lohi
                mid = (lo + hi) // 2
                p = batch_sm[mid] < g
                return jnp.where(p, mid + 1, lo), jnp.where(p, hi, mid)

            lo, _hi = lax.fori_loop(0, 14, bs, (0, NPAD))
            news_scr[pl.ds(g, 1), :] = x_ref[pl.ds(lo, 1), :]
            return 0

        lax.fori_loop(0, NG, root_body, 0)

        hgf = jnp.maximum(
            _mm(hg_scr[...], w0_ref[...], ((1,), (1,))) + b0_ref[...][None, :],
            0.0)
        newsh = jnp.maximum(
            _mm(news_scr[...], wn_ref[...], ((1,), (1,))) + bn_ref[...][None, :],
            0.0)
        w4 = w4_ref[...]
        logit = (_mm(hgf, w4[:, :HID], ((1,), (1,))) +
                 _mm(newsh, w4[:, HID:], ((1,), (1,))) + b4_ref[...][None, :])
        out_ref[...] = jax.nn.sigmoid(logit)


def _final(batch_pad, acc, s2, b3, x_pad, w0, b0, wn, bn, w4, b4):
    grid_spec = pltpu.PrefetchScalarGridSpec(
        num_scalar_prefetch=1,
        grid=(NBLK,),
        in_specs=[
            pl.BlockSpec((2, R, 128), lambda i, b: (0, i, 0)),
            pl.BlockSpec((2, R), lambda i, b: (0, i)),
            pl.BlockSpec((HID,), lambda i, b: (0,)),
            pl.BlockSpec((R, 1), lambda i, b: (i, 0)),
            pl.BlockSpec((NPAD, HID), lambda i, b: (0, 0)),
            pl.BlockSpec((HID, HID), lambda i, b: (0, 0)),
            pl.BlockSpec((HID,), lambda i, b: (0,)),
            pl.BlockSpec((HID, HID), lambda i, b: (0, 0)),
            pl.BlockSpec((HID,), lambda i, b: (0,)),
            pl.BlockSpec((1, 2 * HID), lambda i, b: (0, 0)),
            pl.BlockSpec((1,), lambda i, b: (0,)),
        ],
        out_specs=pl.BlockSpec((NG, 1), lambda i, b: (0, 0)),
        scratch_shapes=[
            pltpu.VMEM((NG, HID), jnp.float32),
            pltpu.VMEM((NG, HID), jnp.float32),
        ],
    )
    return pl.pallas_call(
        _final_body,
        grid_spec=grid_spec,
        out_shape=jax.ShapeDtypeStruct((NG, 1), jnp.float32),
    )(batch_pad, acc, s2, b3, batch_pad[:, None], x_pad,
      w0, b0, wn, bn, w4, b4)


@jax.jit
def _run(x, edge_index, batch, W1, a1s, a1d, b1, W2, a2s, a2d, b2,
         W3, a3s, a3d, b3, Wn, bn, W0, b0, W4, b4):
    loops = jnp.arange(N, dtype=jnp.int32)
    ei = edge_index.astype(jnp.int32)
    src = jnp.concatenate([ei[0], loops])
    dst = jnp.concatenate([ei[1], loops])
    npad_e = EPC - E
    src = jnp.concatenate([src, jnp.zeros((npad_e,), jnp.int32)])
    dst = jnp.concatenate([dst, jnp.full((npad_e,), N, jnp.int32)])
    srcs3d = src.reshape(32, CHA, 128)
    dsts3d = dst.reshape(32, CHA, 128)
    srcsflat = src.reshape(16, 2 * CHA * 128)
    dsts3db = dst.reshape(16, 2 * CHA, 128)

    x_pad = jnp.zeros((NPAD, HID), jnp.float32).at[:N].set(x)
    batch_pad = jnp.concatenate(
        [batch.astype(jnp.int32), jnp.full((NPAD - N,), NG, jnp.int32)])

    h2, asv, adv, mas, mad = _mm_first(x_pad, W1, a1s, a1d)
    acc, s2 = _gat_edges(h2, srcs3d, dsts3d, srcsflat, dsts3db,
                         asv, adv, mas, mad)
    h2, asv, adv, mas, mad = _mm_mid(acc, s2, b1, W2, a2s, a2d)
    acc, s2 = _gat_edges(h2, srcs3d, dsts3d, srcsflat, dsts3db,
                         asv, adv, mas, mad)
    h2, asv, adv, mas, mad = _mm_mid(acc, s2, b2, W3, a3s, a3d)
    acc, s2 = _gat_edges(h2, srcs3d, dsts3d, srcsflat, dsts3db,
                         asv, adv, mas, mad)
    return _final(batch_pad, acc, s2, b3, x_pad, W0, b0, Wn, bn, W4, b4)


def kernel(x, edge_index, batch, W1, a1s, a1d, b1, W2, a2s, a2d, b2,
           W3, a3s, a3d, b3, Wn, bn, W0, b0, W4, b4):
    return _run(x, edge_index, batch, W1, a1s, a1d, b1, W2, a2s, a2d, b2,
                W3, a3s, a3d, b3, Wn, bn, W0, b0, W4, b4)


# speedup vs baseline: 16.9790x; 1.1007x over previous
import dataclasses

import jax
import jax.numpy as jnp
from jax import lax
from jax.experimental import pallas as pl
from jax.experimental.pallas import tpu as pltpu
from jax.experimental.pallas import tpu_sc as plsc

N = 10000
NPAD = 10240
E = 170000
CHA = 42
EPC = 32 * CHA * 128
HID = 256
NG = 64
R = 1024
NBLK = NPAD // R

_HIGH = lax.Precision.HIGHEST


def _mm(x, w, dims):
    return lax.dot_general(x, w, (dims, ((), ())), precision=_HIGH,
                           preferred_element_type=jnp.float32)


def _sc_params():
    cp = pltpu.CompilerParams()
    if "needs_layout_passes" in pltpu.CompilerParams.__dataclass_fields__:
        cp = dataclasses.replace(cp, needs_layout_passes=False)
    return cp


def _mm_first_body(x_ref, w_ref, as_ref, ad_ref,
                   h2_ref, asv_ref, adv_ref, mas_ref, mad_ref):
    i = pl.program_id(0)
    h = _mm(x_ref[...], w_ref[...], ((1,), (1,)))
    h2_ref[0] = h[:, :128]
    h2_ref[1] = h[:, 128:]
    asv = _mm(h, as_ref[...], ((1,), (0,)))
    adv = _mm(h, ad_ref[...], ((1,), (0,)))
    asv_ref[...] = asv
    adv_ref[...] = adv

    @pl.when(i == 0)
    def _():
        mas_ref[...] = jnp.full((16,), -3e38, jnp.float32)
        mad_ref[...] = jnp.full((16,), -3e38, jnp.float32)

    mas_ref[...] = jnp.maximum(mas_ref[...], jnp.max(asv))
    mad_ref[...] = jnp.maximum(mad_ref[...], jnp.max(adv))


def _mm_first(x_pad, w, a_s, a_d):
    return pl.pallas_call(
        _mm_first_body,
        grid=(NBLK,),
        in_specs=[
            pl.BlockSpec((R, HID), lambda i: (i, 0)),
            pl.BlockSpec((HID, HID), lambda i: (0, 0)),
            pl.BlockSpec((HID,), lambda i: (0,)),
            pl.BlockSpec((HID,), lambda i: (0,)),
        ],
        out_specs=[
            pl.BlockSpec((2, R, 128), lambda i: (0, i, 0)),
            pl.BlockSpec((R,), lambda i: (i,)),
            pl.BlockSpec((R,), lambda i: (i,)),
            pl.BlockSpec((16,), lambda i: (0,)),
            pl.BlockSpec((16,), lambda i: (0,)),
        ],
        out_shape=[
            jax.ShapeDtypeStruct((2, NPAD, 128), jnp.float32),
            jax.ShapeDtypeStruct((NPAD,), jnp.float32),
            jax.ShapeDtypeStruct((NPAD,), jnp.float32),
            jax.ShapeDtypeStruct((16,), jnp.float32),
            jax.ShapeDtypeStruct((16,), jnp.float32),
        ],
    )(x_pad, w, a_s, a_d)


def _mm_mid_body(acc_ref, s_ref, b_ref, w_ref, as_ref, ad_ref,
                 h2_ref, asv_ref, adv_ref, mas_ref, mad_ref):
    i = pl.program_id(0)
    s = s_ref[0] + s_ref[1]
    rcp = 1.0 / jnp.maximum(s, 1e-30)
    b = b_ref[...]
    x0 = jnp.maximum(acc_ref[0] * rcp[:, None] + b[None, :128], 0.0)
    x1 = jnp.maximum(acc_ref[1] * rcp[:, None] + b[None, 128:], 0.0)
    w = w_ref[...]
    h = _mm(x0, w[:, :128], ((1,), (1,))) + _mm(x1, w[:, 128:], ((1,), (1,)))
    h2_ref[0] = h[:, :128]
    h2_ref[1] = h[:, 128:]
    asv = _mm(h, as_ref[...], ((1,), (0,)))
    adv = _mm(h, ad_ref[...], ((1,), (0,)))
    asv_ref[...] = asv
    adv_ref[...] = adv

    @pl.when(i == 0)
    def _():
        mas_ref[...] = jnp.full((16,), -3e38, jnp.float32)
        mad_ref[...] = jnp.full((16,), -3e38, jnp.float32)

    mas_ref[...] = jnp.maximum(mas_ref[...], jnp.max(asv))
    mad_ref[...] = jnp.maximum(mad_ref[...], jnp.max(adv))


def _mm_mid(acc, s2, b, w, a_s, a_d):
    return pl.pallas_call(
        _mm_mid_body,
        grid=(NBLK,),
        in_specs=[
            pl.BlockSpec((2, R, 128), lambda i: (0, i, 0)),
            pl.BlockSpec((2, R), lambda i: (0, i)),
            pl.BlockSpec((HID,), lambda i: (0,)),
            pl.BlockSpec((HID, HID), lambda i: (0, 0)),
            pl.BlockSpec((HID,), lambda i: (0,)),
            pl.BlockSpec((HID,), lambda i: (0,)),
        ],
        out_specs=[
            pl.BlockSpec((2, R, 128), lambda i: (0, i, 0)),
            pl.BlockSpec((R,), lambda i: (i,)),
            pl.BlockSpec((R,), lambda i: (i,)),
            pl.BlockSpec((16,), lambda i: (0,)),
            pl.BlockSpec((16,), lambda i: (0,)),
        ],
        out_shape=[
            jax.ShapeDtypeStruct((2, NPAD, 128), jnp.float32),
            jax.ShapeDtypeStruct((NPAD,), jnp.float32),
            jax.ShapeDtypeStruct((NPAD,), jnp.float32),
            jax.ShapeDtypeStruct((16,), jnp.float32),
            jax.ShapeDtypeStruct((16,), jnp.float32),
        ],
    )(acc, s2, b, w, a_s, a_d)


def _sc_soft_body(srcs_hbm, dsts_hbm, asrc_hbm, adst_hbm, mas_hbm, mad_hbm,
                  ex_hbm, sout_hbm,
                  asrc_t, adst_t, src_t, dst_t, ex_t, mas_t, mad_t, z_t, s_sh):
    c = lax.axis_index("c")
    sid = lax.axis_index("s")
    w = 2 * sid + c

    pltpu.sync_copy(srcs_hbm.at[w], src_t)
    pltpu.sync_copy(dsts_hbm.at[w], dst_t)
    pltpu.sync_copy(asrc_hbm, asrc_t)
    pltpu.sync_copy(adst_hbm, adst_t)
    pltpu.sync_copy(mas_hbm, mas_t)
    pltpu.sync_copy(mad_hbm, mad_t)

    zv = jnp.zeros((16,), jnp.float32)

    @pl.loop(0, 640, step=16)
    def _(i):
        z_t[pl.ds(i, 16)] = zv

    pltpu.sync_copy(z_t, s_sh.at[pl.ds(sid * 640, 640)])
    plsc.subcore_barrier()

    t_m = mas_t[...] + mad_t[...]
    m_vec = jnp.maximum(t_m, 0.2 * t_m)

    @pl.loop(0, CHA)
    def _(j):
        for k in range(8):
            s16 = src_t[j, pl.ds(k * 16, 16)]
            d16 = dst_t[j, pl.ds(k * 16, 16)]
            av = plsc.load_gather(asrc_t, [s16])
            bv = plsc.load_gather(adst_t, [d16])
            t = av + bv
            e = jnp.maximum(t, 0.2 * t)
            ex_t[j, pl.ds(k * 16, 16)] = jnp.exp(e - m_vec)
        pltpu.sync_copy(ex_t.at[j], s_sh.at[dst_t.at[j]], add=True)

    pltpu.sync_copy(ex_t, ex_hbm.at[w])
    plsc.subcore_barrier()
    pltpu.sync_copy(s_sh.at[pl.ds(sid * 640, 640)],
                    sout_hbm.at[c].at[pl.ds(sid * 640, 640)])


def _sc_soft(srcs3d, dsts3d, asv, adv, mas, mad):
    mesh = plsc.VectorSubcoreMesh(core_axis_name="c", subcore_axis_name="s")
    f = pl.kernel(
        _sc_soft_body,
        compiler_params=_sc_params(),
        out_type=[
            jax.ShapeDtypeStruct((32, CHA, 128), jnp.float32),
            jax.ShapeDtypeStruct((2, NPAD), jnp.float32),
        ],
        mesh=mesh,
        scratch_types=[
            pltpu.VMEM((NPAD,), jnp.float32),
            pltpu.VMEM((NPAD,), jnp.float32),
            pltpu.VMEM((CHA, 128), jnp.int32),
            pltpu.VMEM((CHA, 128), jnp.int32),
            pltpu.VMEM((CHA, 128), jnp.float32),
            pltpu.VMEM((16,), jnp.float32),
            pltpu.VMEM((16,), jnp.float32),
            pltpu.VMEM((640,), jnp.float32),
            pltpu.VMEM_SHARED((NPAD,), jnp.float32),
        ],
    )
    return f(srcs3d, dsts3d, asv, adv, mas, mad)


def _sc_agg_body(h2_hbm, srcs_hbm, dsts_hbm, ex_hbm, acc_hbm,
                 dst_t, s0_t, s1_t, ex0_t, ex1_t, rows0_t, rows1_t,
                 semg0, semg1, seme0, seme1, semi0, semi1, acc_sh):
    s_t = (s0_t, s1_t)
    exj_t = (ex0_t, ex1_t)
    rows_t = (rows0_t, rows1_t)
    semg = (semg0, semg1)
    seme = (seme0, seme1)
    semi = (semi0, semi1)
    c = lax.axis_index("c")
    sid = lax.axis_index("s")

    pltpu.sync_copy(dsts_hbm.at[sid], dst_t)
    srcv = srcs_hbm.at[sid]
    exv = ex_hbm.at[sid]
    hsel = h2_hbm.at[c]
    last = 2 * CHA - 1

    zv = jnp.zeros((16,), jnp.float32)

    @pl.loop(0, 128)
    def _(r):
        for k in range(8):
            rows_t[0][r, pl.ds(k * 16, 16)] = zv

    for k in range(5):
        pltpu.sync_copy(rows_t[0], acc_sh.at[pl.ds(sid * 640 + k * 128, 128)])
    plsc.subcore_barrier()

    for p in range(2):
        pltpu.sync_copy(srcv.at[pl.ds(p * 128, 128)], s_t[p])
        pltpu.async_copy(hsel.at[s_t[p]], rows_t[p], semg[p])
        pltpu.async_copy(exv.at[pl.ds(p * 128, 128)], exj_t[p], seme[p])

    def scale_scatter(j, p):
        pltpu.make_async_copy(hsel.at[s_t[p]], rows_t[p], semg[p]).wait()
        nj = jnp.minimum(j + 2, last)
        pltpu.async_copy(srcv.at[pl.ds(nj * 128, 128)], s_t[p], semi[p])
        pltpu.make_async_copy(exv.at[pl.ds(0, 128)], exj_t[p], seme[p]).wait()

        @plsc.parallel_loop(0, 128, unroll=4)
        def _(r):
            av = plsc.load_gather(exj_t[p], [jnp.full((16,), r, jnp.int32)])
            for k in range(8):
                rows_t[p][r, pl.ds(k * 16, 16)] = (
                    rows_t[p][r, pl.ds(k * 16, 16)] * av)

        pltpu.async_copy(exv.at[pl.ds(nj * 128, 128)], exj_t[p], seme[p])
        pltpu.sync_copy(rows_t[p], acc_sh.at[dst_t.at[j]], add=True)
        pltpu.make_async_copy(srcv.at[pl.ds(0, 128)], s_t[p], semi[p]).wait()
        pltpu.async_copy(hsel.at[s_t[p]], rows_t[p], semg[p])

    @pl.loop(0, CHA)
    def _(j2):
        scale_scatter(2 * j2, 0)
        scale_scatter(2 * j2 + 1, 1)

    for p in range(2):
        pltpu.make_async_copy(hsel.at[s_t[p]], rows_t[p], semg[p]).wait()
        pltpu.make_async_copy(exv.at[pl.ds(0, 128)], exj_t[p], seme[p]).wait()

    plsc.subcore_barrier()
    pltpu.sync_copy(acc_sh.at[pl.ds(sid * 640, 640)],
                    acc_hbm.at[c].at[pl.ds(sid * 640, 640)])


def _sc_agg(h2, srcs3db, dsts3db, exb):
    mesh = plsc.VectorSubcoreMesh(core_axis_name="c", subcore_axis_name="s")
    f = pl.kernel(
        _sc_agg_body,
        compiler_params=_sc_params(),
        out_type=jax.ShapeDtypeStruct((2, NPAD, 128), jnp.float32),
        mesh=mesh,
        scratch_types=[
            pltpu.VMEM((2 * CHA, 128), jnp.int32),
            pltpu.VMEM((128,), jnp.int32),
            pltpu.VMEM((128,), jnp.int32),
            pltpu.VMEM((128,), jnp.float32),
            pltpu.VMEM((128,), jnp.float32),
            pltpu.VMEM((128, 128), jnp.float32),
            pltpu.VMEM((128, 128), jnp.float32),
            pltpu.SemaphoreType.DMA,
            pltpu.SemaphoreType.DMA,
            pltpu.SemaphoreType.DMA,
            pltpu.SemaphoreType.DMA,
            pltpu.SemaphoreType.DMA,
            pltpu.SemaphoreType.DMA,
            pltpu.VMEM_SHARED((NPAD, 128), jnp.float32),
        ],
    )
    return f(h2, srcs3db, dsts3db, exb)


def _gat_edges(h2, srcs3d, dsts3d, srcsflat, dsts3db, asv, adv, mas, mad):
    ex, s2 = _sc_soft(srcs3d, dsts3d, asv, adv, mas, mad)
    acc = _sc_agg(h2, srcsflat, dsts3db, ex.reshape(16, 2 * CHA * 128))
    return acc, s2


def _final_body(batch_sm, acc_ref, s_ref, b3_ref, bv_ref, x_ref,
                w0_ref, b0_ref, wn_ref, bn_ref, w4_ref, b4_ref,
                out_ref, hg_scr, news_scr):
    i = pl.program_id(0)

    @pl.when(i == 0)
    def _():
        hg_scr[...] = jnp.zeros((NG, HID), jnp.float32)

    s = s_ref[0] + s_ref[1]
    rcp = 1.0 / jnp.maximum(s, 1e-30)
    b3 = b3_ref[...]
    h0 = jnp.maximum(acc_ref[0] * rcp[:, None] + b3[None, :128], 0.0)
    h1 = jnp.maximum(acc_ref[1] * rcp[:, None] + b3[None, 128:], 0.0)
    h3 = jnp.concatenate([h0, h1], axis=1)
    bv = bv_ref[...]

    g_lo = batch_sm[i * R]
    g_hi = jnp.minimum(batch_sm[i * R + R - 1], NG - 1)

    def seg_body(g, _):
        mask = bv == g
        mx = jnp.max(jnp.where(mask, h3, 0.0), axis=0, keepdims=True)
        hg_scr[pl.ds(g, 1), :] = jnp.maximum(hg_scr[pl.ds(g, 1), :], mx)
        return 0

    lax.fori_loop(g_lo, g_hi + 1, seg_body, 0)

    @pl.when(i == NBLK - 1)
    def _():
        def root_body(g, _):
            def bs(_, lohi):
                lo, hi = lohi
                mid = (lo + hi) // 2
                p = batch_sm[mid] < g
                return jnp.where(p, mid + 1, lo), jnp.where(p, hi, mid)

            lo, _hi = lax.fori_loop(0, 14, bs, (0, NPAD))
            news_scr[pl.ds(g, 1), :] = x_ref[pl.ds(lo, 1), :]
            return 0

        lax.fori_loop(0, NG, root_body, 0)

        hgf = jnp.maximum(
            _mm(hg_scr[...], w0_ref[...], ((1,), (1,))) + b0_ref[...][None, :],
            0.0)
        newsh = jnp.maximum(
            _mm(news_scr[...], wn_ref[...], ((1,), (1,))) + bn_ref[...][None, :],
            0.0)
        w4 = w4_ref[...]
        logit = (_mm(hgf, w4[:, :HID], ((1,), (1,))) +
                 _mm(newsh, w4[:, HID:], ((1,), (1,))) + b4_ref[...][None, :])
        out_ref[...] = jax.nn.sigmoid(logit)


def _final(batch_pad, acc, s2, b3, x_pad, w0, b0, wn, bn, w4, b4):
    grid_spec = pltpu.PrefetchScalarGridSpec(
        num_scalar_prefetch=1,
        grid=(NBLK,),
        in_specs=[
            pl.BlockSpec((2, R, 128), lambda i, b: (0, i, 0)),
            pl.BlockSpec((2, R), lambda i, b: (0, i)),
            pl.BlockSpec((HID,), lambda i, b: (0,)),
            pl.BlockSpec((R, 1), lambda i, b: (i, 0)),
            pl.BlockSpec((NPAD, HID), lambda i, b: (0, 0)),
            pl.BlockSpec((HID, HID), lambda i, b: (0, 0)),
            pl.BlockSpec((HID,), lambda i, b: (0,)),
            pl.BlockSpec((HID, HID), lambda i, b: (0, 0)),
            pl.BlockSpec((HID,), lambda i, b: (0,)),
            pl.BlockSpec((1, 2 * HID), lambda i, b: (0, 0)),
            pl.BlockSpec((1,), lambda i, b: (0,)),
        ],
        out_specs=pl.BlockSpec((NG, 1), lambda i, b: (0, 0)),
        scratch_shapes=[
            pltpu.VMEM((NG, HID), jnp.float32),
            pltpu.VMEM((NG, HID), jnp.float32),
        ],
    )
    return pl.pallas_call(
        _final_body,
        grid_spec=grid_spec,
        out_shape=jax.ShapeDtypeStruct((NG, 1), jnp.float32),
    )(batch_pad, acc, s2, b3, batch_pad[:, None], x_pad,
      w0, b0, wn, bn, w4, b4)


@jax.jit
def _run(x, edge_index, batch, W1, a1s, a1d, b1, W2, a2s, a2d, b2,
         W3, a3s, a3d, b3, Wn, bn, W0, b0, W4, b4):
    loops = jnp.arange(N, dtype=jnp.int32)
    ei = edge_index.astype(jnp.int32)
    src = jnp.concatenate([ei[0], loops])
    dst = jnp.concatenate([ei[1], loops])
    npad_e = EPC - E
    src = jnp.concatenate([src, jnp.zeros((npad_e,), jnp.int32)])
    dst = jnp.concatenate([dst, jnp.full((npad_e,), N, jnp.int32)])
    srcs3d = src.reshape(32, CHA, 128)
    dsts3d = dst.reshape(32, CHA, 128)
    srcsflat = src.reshape(16, 2 * CHA * 128)
    dsts3db = dst.reshape(16, 2 * CHA, 128)

    x_pad = jnp.zeros((NPAD, HID), jnp.float32).at[:N].set(x)
    batch_pad = jnp.concatenate(
        [batch.astype(jnp.int32), jnp.full((NPAD - N,), NG, jnp.int32)])

    h2, asv, adv, mas, mad = _mm_first(x_pad, W1, a1s, a1d)
    acc, s2 = _gat_edges(h2, srcs3d, dsts3d, srcsflat, dsts3db,
                         asv, adv, mas, mad)
    h2, asv, adv, mas, mad = _mm_mid(acc, s2, b1, W2, a2s, a2d)
    acc, s2 = _gat_edges(h2, srcs3d, dsts3d, srcsflat, dsts3db,
                         asv, adv, mas, mad)
    h2, asv, adv, mas, mad = _mm_mid(acc, s2, b2, W3, a3s, a3d)
    acc, s2 = _gat_edges(h2, srcs3d, dsts3d, srcsflat, dsts3db,
                         asv, adv, mas, mad)
    return _final(batch_pad, acc, s2, b3, x_pad, W0, b0, Wn, bn, W4, b4)


def kernel(x, edge_index, batch, W1, a1s, a1d, b1, W2, a2s, a2d, b2,
           W3, a3s, a3d, b3, Wn, bn, W0, b0, W4, b4):
    return _run(x, edge_index, batch, W1, a1s, a1d, b1, W2, a2s, a2d, b2,
                W3, a3s, a3d, b3, Wn, bn, W0, b0, W4, b4)


# speedup vs baseline: 17.6013x; 1.0367x over previous
import dataclasses

import jax
import jax.numpy as jnp
from jax import lax
from jax.experimental import pallas as pl
from jax.experimental.pallas import tpu as pltpu
from jax.experimental.pallas import tpu_sc as plsc

N = 10000
NPAD = 10240
E = 170000
CHA = 42
EPC = 32 * CHA * 128
CK = 128
NCHK = EPC // 16 // CK
NACC = 10048
HID = 256
NG = 64
R = 1024
NBLK = NPAD // R

_HIGH = lax.Precision.HIGHEST


def _mm(x, w, dims):
    return lax.dot_general(x, w, (dims, ((), ())), precision=_HIGH,
                           preferred_element_type=jnp.float32)


def _sc_params():
    cp = pltpu.CompilerParams()
    if "needs_layout_passes" in pltpu.CompilerParams.__dataclass_fields__:
        cp = dataclasses.replace(cp, needs_layout_passes=False)
    return cp


def _mm_first_body(x_ref, w_ref, as_ref, ad_ref,
                   h2_ref, asv_ref, adv_ref, mas_ref, mad_ref):
    i = pl.program_id(0)
    h = _mm(x_ref[...], w_ref[...], ((1,), (1,)))
    h2_ref[0] = h[:, :128]
    h2_ref[1] = h[:, 128:]
    asv = _mm(h, as_ref[...], ((1,), (0,)))
    adv = _mm(h, ad_ref[...], ((1,), (0,)))
    asv_ref[...] = asv
    adv_ref[...] = adv

    @pl.when(i == 0)
    def _():
        mas_ref[...] = jnp.full((16,), -3e38, jnp.float32)
        mad_ref[...] = jnp.full((16,), -3e38, jnp.float32)

    mas_ref[...] = jnp.maximum(mas_ref[...], jnp.max(asv))
    mad_ref[...] = jnp.maximum(mad_ref[...], jnp.max(adv))


def _mm_first(x_pad, w, a_s, a_d):
    return pl.pallas_call(
        _mm_first_body,
        grid=(NBLK,),
        in_specs=[
            pl.BlockSpec((R, HID), lambda i: (i, 0)),
            pl.BlockSpec((HID, HID), lambda i: (0, 0)),
            pl.BlockSpec((HID,), lambda i: (0,)),
            pl.BlockSpec((HID,), lambda i: (0,)),
        ],
        out_specs=[
            pl.BlockSpec((2, R, 128), lambda i: (0, i, 0)),
            pl.BlockSpec((R,), lambda i: (i,)),
            pl.BlockSpec((R,), lambda i: (i,)),
            pl.BlockSpec((16,), lambda i: (0,)),
            pl.BlockSpec((16,), lambda i: (0,)),
        ],
        out_shape=[
            jax.ShapeDtypeStruct((2, NPAD, 128), jnp.float32),
            jax.ShapeDtypeStruct((NPAD,), jnp.float32),
            jax.ShapeDtypeStruct((NPAD,), jnp.float32),
            jax.ShapeDtypeStruct((16,), jnp.float32),
            jax.ShapeDtypeStruct((16,), jnp.float32),
        ],
    )(x_pad, w, a_s, a_d)


def _mm_mid_body(acc_ref, s_ref, b_ref, w_ref, as_ref, ad_ref,
                 h2_ref, asv_ref, adv_ref, mas_ref, mad_ref):
    i = pl.program_id(0)
    s = s_ref[0] + s_ref[1]
    rcp = 1.0 / jnp.maximum(s, 1e-30)
    b = b_ref[...]
    x0 = jnp.maximum(acc_ref[0] * rcp[:, None] + b[None, :128], 0.0)
    x1 = jnp.maximum(acc_ref[1] * rcp[:, None] + b[None, 128:], 0.0)
    w = w_ref[...]
    h = _mm(x0, w[:, :128], ((1,), (1,))) + _mm(x1, w[:, 128:], ((1,), (1,)))
    h2_ref[0] = h[:, :128]
    h2_ref[1] = h[:, 128:]
    asv = _mm(h, as_ref[...], ((1,), (0,)))
    adv = _mm(h, ad_ref[...], ((1,), (0,)))
    asv_ref[...] = asv
    adv_ref[...] = adv

    @pl.when(i == 0)
    def _():
        mas_ref[...] = jnp.full((16,), -3e38, jnp.float32)
        mad_ref[...] = jnp.full((16,), -3e38, jnp.float32)

    mas_ref[...] = jnp.maximum(mas_ref[...], jnp.max(asv))
    mad_ref[...] = jnp.maximum(mad_ref[...], jnp.max(adv))


def _mm_mid(acc, s2, b, w, a_s, a_d):
    return pl.pallas_call(
        _mm_mid_body,
        grid=(NBLK,),
        in_specs=[
            pl.BlockSpec((2, R, 128), lambda i: (0, i, 0)),
            pl.BlockSpec((2, R), lambda i: (0, i)),
            pl.BlockSpec((HID,), lambda i: (0,)),
            pl.BlockSpec((HID, HID), lambda i: (0, 0)),
            pl.BlockSpec((HID,), lambda i: (0,)),
            pl.BlockSpec((HID,), lambda i: (0,)),
        ],
        out_specs=[
            pl.BlockSpec((2, R, 128), lambda i: (0, i, 0)),
            pl.BlockSpec((R,), lambda i: (i,)),
            pl.BlockSpec((R,), lambda i: (i,)),
            pl.BlockSpec((16,), lambda i: (0,)),
            pl.BlockSpec((16,), lambda i: (0,)),
        ],
        out_shape=[
            jax.ShapeDtypeStruct((2, NPAD, 128), jnp.float32),
            jax.ShapeDtypeStruct((NPAD,), jnp.float32),
            jax.ShapeDtypeStruct((NPAD,), jnp.float32),
            jax.ShapeDtypeStruct((16,), jnp.float32),
            jax.ShapeDtypeStruct((16,), jnp.float32),
        ],
    )(acc, s2, b, w, a_s, a_d)


def _sc_soft_body(srcs_hbm, dsts_hbm, asrc_hbm, adst_hbm, mas_hbm, mad_hbm,
                  ex_hbm, sout_hbm,
                  asrc_t, adst_t, src_t, dst_t, ex_t, mas_t, mad_t, z_t, s_sh):
    c = lax.axis_index("c")
    sid = lax.axis_index("s")
    w = 2 * sid + c

    pltpu.sync_copy(srcs_hbm.at[w], src_t)
    pltpu.sync_copy(dsts_hbm.at[w], dst_t)
    pltpu.sync_copy(asrc_hbm, asrc_t)
    pltpu.sync_copy(adst_hbm, adst_t)
    pltpu.sync_copy(mas_hbm, mas_t)
    pltpu.sync_copy(mad_hbm, mad_t)

    zv = jnp.zeros((16,), jnp.float32)

    @pl.loop(0, 640, step=16)
    def _(i):
        z_t[pl.ds(i, 16)] = zv

    pltpu.sync_copy(z_t, s_sh.at[pl.ds(sid * 640, 640)])
    plsc.subcore_barrier()

    t_m = mas_t[...] + mad_t[...]
    m_vec = jnp.maximum(t_m, 0.2 * t_m)

    @pl.loop(0, CHA)
    def _(j):
        for k in range(8):
            s16 = src_t[j, pl.ds(k * 16, 16)]
            d16 = dst_t[j, pl.ds(k * 16, 16)]
            av = plsc.load_gather(asrc_t, [s16])
            bv = plsc.load_gather(adst_t, [d16])
            t = av + bv
            e = jnp.maximum(t, 0.2 * t)
            ex_t[j, pl.ds(k * 16, 16)] = jnp.exp(e - m_vec)
        pltpu.sync_copy(ex_t.at[j], s_sh.at[dst_t.at[j]], add=True)

    pltpu.sync_copy(ex_t, ex_hbm.at[w])
    plsc.subcore_barrier()
    pltpu.sync_copy(s_sh.at[pl.ds(sid * 640, 640)],
                    sout_hbm.at[c].at[pl.ds(sid * 640, 640)])


def _sc_soft(srcs3d, dsts3d, asv, adv, mas, mad):
    mesh = plsc.VectorSubcoreMesh(core_axis_name="c", subcore_axis_name="s")
    f = pl.kernel(
        _sc_soft_body,
        compiler_params=_sc_params(),
        out_type=[
            jax.ShapeDtypeStruct((32, CHA, 128), jnp.float32),
            jax.ShapeDtypeStruct((2, NPAD), jnp.float32),
        ],
        mesh=mesh,
        scratch_types=[
            pltpu.VMEM((NPAD,), jnp.float32),
            pltpu.VMEM((NPAD,), jnp.float32),
            pltpu.VMEM((CHA, 128), jnp.int32),
            pltpu.VMEM((CHA, 128), jnp.int32),
            pltpu.VMEM((CHA, 128), jnp.float32),
            pltpu.VMEM((16,), jnp.float32),
            pltpu.VMEM((16,), jnp.float32),
            pltpu.VMEM((640,), jnp.float32),
            pltpu.VMEM_SHARED((NPAD,), jnp.float32),
        ],
    )
    return f(srcs3d, dsts3d, asv, adv, mas, mad)


def _sc_agg_body(h2_hbm, srcs_hbm, dsts_hbm, ex_hbm, acc_hbm,
                 s0_t, s1_t, s2_t, d0_t, d1_t, d2_t, ex0_t, ex1_t, ex2_t,
                 rows0_t, rows1_t, rows2_t,
                 semg0, semg1, semg2, seme0, seme1, seme2,
                 semi0, semi1, semi2, semd0, semd1, semd2,
                 sems0, sems1, sems2, acc_sh):
    s_t = (s0_t, s1_t, s2_t)
    d_t = (d0_t, d1_t, d2_t)
    exj_t = (ex0_t, ex1_t, ex2_t)
    rows_t = (rows0_t, rows1_t, rows2_t)
    semg = (semg0, semg1, semg2)
    seme = (seme0, seme1, seme2)
    semi = (semi0, semi1, semi2)
    semd = (semd0, semd1, semd2)
    sems = (sems0, sems1, sems2)
    c = lax.axis_index("c")
    sid = lax.axis_index("s")

    srcv = srcs_hbm.at[sid]
    dstv = dsts_hbm.at[sid]
    exv = ex_hbm.at[sid]
    hsel = h2_hbm.at[c]
    last = NCHK - 1

    zv = jnp.zeros((16,), jnp.float32)

    @pl.loop(0, CK)
    def _(r):
        for k in range(8):
            rows0_t[r, pl.ds(k * 16, 16)] = zv

    @pl.when(sid < 15)
    def _():
        for k in range(4):
            pltpu.sync_copy(rows0_t,
                            acc_sh.at[pl.ds(sid * 632 + k * CK, CK)])
        pltpu.sync_copy(rows0_t.at[pl.ds(0, 120)],
                        acc_sh.at[pl.ds(sid * 632 + 512, 120)])

    @pl.when(sid == 15)
    def _():
        for k in range(4):
            pltpu.sync_copy(rows0_t,
                            acc_sh.at[pl.ds(15 * 632 + k * CK, CK)])
        pltpu.sync_copy(rows0_t.at[pl.ds(0, 56)],
                        acc_sh.at[pl.ds(15 * 632 + 512, 56)])

    plsc.subcore_barrier()

    for p in range(2):
        pltpu.sync_copy(srcv.at[pl.ds(p * CK, CK)], s_t[p])
        pltpu.async_copy(hsel.at[s_t[p]], rows_t[p], semg[p])
        pltpu.async_copy(exv.at[pl.ds(p * CK, CK)], exj_t[p], seme[p])
        pltpu.async_copy(dstv.at[pl.ds(p * CK, CK)], d_t[p], semd[p])

    def step(i, p, first):
        q = (p + 2) % 3
        nj = jnp.minimum(i + 2, last)
        pltpu.async_copy(srcv.at[pl.ds(nj * CK, CK)], s_t[q], semi[q])
        pltpu.async_copy(exv.at[pl.ds(nj * CK, CK)], exj_t[q], seme[q])
        pltpu.make_async_copy(hsel.at[s_t[p]], rows_t[p], semg[p]).wait()
        pltpu.make_async_copy(exv.at[pl.ds(0, CK)], exj_t[p], seme[p]).wait()

        @plsc.parallel_loop(0, CK, unroll=4)
        def _(r):
            av = plsc.load_gather(exj_t[p], [jnp.full((16,), r, jnp.int32)])
            for k in range(8):
                rows_t[p][r, pl.ds(k * 16, 16)] = (
                    rows_t[p][r, pl.ds(k * 16, 16)] * av)

        pltpu.make_async_copy(dstv.at[pl.ds(0, CK)], d_t[p], semd[p]).wait()
        pltpu.async_copy(rows_t[p], acc_sh.at[d_t[p]], sems[p], add=True)

        if first:
            @pl.when(i > 0)
            def _():
                pltpu.make_async_copy(rows_t[q], acc_sh.at[d_t[q]],
                                      sems[q]).wait()
        else:
            pltpu.make_async_copy(rows_t[q], acc_sh.at[d_t[q]],
                                  sems[q]).wait()

        pltpu.async_copy(dstv.at[pl.ds(nj * CK, CK)], d_t[q], semd[q])
        pltpu.make_async_copy(srcv.at[pl.ds(0, CK)], s_t[q], semi[q]).wait()
        pltpu.async_copy(hsel.at[s_t[q]], rows_t[q], semg[q])

    @pl.loop(0, NCHK // 3)
    def _(o):
        i0 = 3 * o
        step(i0, 0, True)
        step(i0 + 1, 1, False)
        step(i0 + 2, 2, False)

    lp = (NCHK - 1) % 3
    pltpu.make_async_copy(rows_t[lp], acc_sh.at[d_t[lp]], sems[lp]).wait()
    for q in ((lp + 1) % 3, (lp + 2) % 3):
        pltpu.make_async_copy(hsel.at[s_t[q]], rows_t[q], semg[q]).wait()
        pltpu.make_async_copy(exv.at[pl.ds(0, CK)], exj_t[q], seme[q]).wait()
        pltpu.make_async_copy(dstv.at[pl.ds(0, CK)], d_t[q], semd[q]).wait()

    plsc.subcore_barrier()

    @pl.when(sid < 15)
    def _():
        pltpu.sync_copy(acc_sh.at[pl.ds(sid * 632, 632)],
                        acc_hbm.at[c].at[pl.ds(sid * 632, 632)])

    @pl.when(sid == 15)
    def _():
        pltpu.sync_copy(acc_sh.at[pl.ds(15 * 632, 568)],
                        acc_hbm.at[c].at[pl.ds(15 * 632, 568)])
        @pl.loop(0, CK)
        def _(r):
            for k in range(8):
                rows0_t[r, pl.ds(k * 16, 16)] = zv

        pltpu.sync_copy(rows0_t, acc_hbm.at[c].at[pl.ds(NACC, CK)])
        pltpu.sync_copy(rows0_t.at[pl.ds(0, 64)],
                        acc_hbm.at[c].at[pl.ds(NACC + CK, 64)])


def _sc_agg(h2, srcs3db, dsts3db, exb):
    mesh = plsc.VectorSubcoreMesh(core_axis_name="c", subcore_axis_name="s")
    f = pl.kernel(
        _sc_agg_body,
        compiler_params=_sc_params(),
        out_type=jax.ShapeDtypeStruct((2, NPAD, 128), jnp.float32),
        mesh=mesh,
        scratch_types=(
            [pltpu.VMEM((CK,), jnp.int32)] * 3
            + [pltpu.VMEM((CK,), jnp.int32)] * 3
            + [pltpu.VMEM((CK,), jnp.float32)] * 3
            + [pltpu.VMEM((CK, 128), jnp.float32)] * 3
            + [pltpu.SemaphoreType.DMA] * 15
            + [pltpu.VMEM_SHARED((NACC, 128), jnp.float32)]
        ),
    )
    return f(h2, srcs3db, dsts3db, exb)


def _gat_edges(h2, srcs3d, dsts3d, srcsflat, dstsflat, asv, adv, mas, mad):
    ex, s2 = _sc_soft(srcs3d, dsts3d, asv, adv, mas, mad)
    acc = _sc_agg(h2, srcsflat, dstsflat, ex.reshape(16, 2 * CHA * 128))
    return acc, s2


def _final_body(batch_sm, acc_ref, s_ref, b3_ref, bv_ref, x_ref,
                w0_ref, b0_ref, wn_ref, bn_ref, w4_ref, b4_ref,
                out_ref, hg_scr, news_scr):
    i = pl.program_id(0)

    @pl.when(i == 0)
    def _():
        hg_scr[...] = jnp.zeros((NG, HID), jnp.float32)

    s = s_ref[0] + s_ref[1]
    rcp = 1.0 / jnp.maximum(s, 1e-30)
    b3 = b3_ref[...]
    h0 = jnp.maximum(acc_ref[0] * rcp[:, None] + b3[None, :128], 0.0)
    h1 = jnp.maximum(acc_ref[1] * rcp[:, None] + b3[None, 128:], 0.0)
    h3 = jnp.concatenate([h0, h1], axis=1)
    bv = bv_ref[...]

    g_lo = batch_sm[i * R]
    g_hi = jnp.minimum(batch_sm[i * R + R - 1], NG - 1)

    def seg_body(g, _):
        mask = bv == g
        mx = jnp.max(jnp.where(mask, h3, 0.0), axis=0, keepdims=True)
        hg_scr[pl.ds(g, 1), :] = jnp.maximum(hg_scr[pl.ds(g, 1), :], mx)
        return 0

    lax.fori_loop(g_lo, g_hi + 1, seg_body, 0)

    @pl.when(i == NBLK - 1)
    def _():
        def root_body(g, _):
            def bs(_, lohi):
                lo, hi = lohi
                mid = (lo + hi) // 2
                p = batch_sm[mid] < g
                return jnp.where(p, mid + 1, lo), jnp.where(p, hi, mid)

            lo, _hi = lax.fori_loop(0, 14, bs, (0, NPAD))
            news_scr[pl.ds(g, 1), :] = x_ref[pl.ds(lo, 1), :]
            return 0

        lax.fori_loop(0, NG, root_body, 0)

        hgf = jnp.maximum(
            _mm(hg_scr[...], w0_ref[...], ((1,), (1,))) + b0_ref[...][None, :],
            0.0)
        newsh = jnp.maximum(
            _mm(news_scr[...], wn_ref[...], ((1,), (1,))) + bn_ref[...][None, :],
            0.0)
        w4 = w4_ref[...]
        logit = (_mm(hgf, w4[:, :HID], ((1,), (1,))) +
                 _mm(newsh, w4[:, HID:], ((1,), (1,))) + b4_ref[...][None, :])
        out_ref[...] = jax.nn.sigmoid(logit)


def _final(batch_pad, acc, s2, b3, x_pad, w0, b0, wn, bn, w4, b4):
    grid_spec = pltpu.PrefetchScalarGridSpec(
        num_scalar_prefetch=1,
        grid=(NBLK,),
        in_specs=[
            pl.BlockSpec((2, R, 128), lambda i, b: (0, i, 0)),
            pl.BlockSpec((2, R), lambda i, b: (0, i)),
            pl.BlockSpec((HID,), lambda i, b: (0,)),
            pl.BlockSpec((R, 1), lambda i, b: (i, 0)),
            pl.BlockSpec((NPAD, HID), lambda i, b: (0, 0)),
            pl.BlockSpec((HID, HID), lambda i, b: (0, 0)),
            pl.BlockSpec((HID,), lambda i, b: (0,)),
            pl.BlockSpec((HID, HID), lambda i, b: (0, 0)),
            pl.BlockSpec((HID,), lambda i, b: (0,)),
            pl.BlockSpec((1, 2 * HID), lambda i, b: (0, 0)),
            pl.BlockSpec((1,), lambda i, b: (0,)),
        ],
        out_specs=pl.BlockSpec((NG, 1), lambda i, b: (0, 0)),
        scratch_shapes=[
            pltpu.VMEM((NG, HID), jnp.float32),
            pltpu.VMEM((NG, HID), jnp.float32),
        ],
    )
    return pl.pallas_call(
        _final_body,
        grid_spec=grid_spec,
        out_shape=jax.ShapeDtypeStruct((NG, 1), jnp.float32),
    )(batch_pad, acc, s2, b3, batch_pad[:, None], x_pad,
      w0, b0, wn, bn, w4, b4)


@jax.jit
def _run(x, edge_index, batch, W1, a1s, a1d, b1, W2, a2s, a2d, b2,
         W3, a3s, a3d, b3, Wn, bn, W0, b0, W4, b4):
    loops = jnp.arange(N, dtype=jnp.int32)
    ei = edge_index.astype(jnp.int32)
    src = jnp.concatenate([ei[0], loops])
    dst = jnp.concatenate([ei[1], loops])
    npad_e = EPC - E
    src = jnp.concatenate([src, jnp.zeros((npad_e,), jnp.int32)])
    dst = jnp.concatenate([dst, jnp.full((npad_e,), N, jnp.int32)])
    srcs3d = src.reshape(32, CHA, 128)
    dsts3d = dst.reshape(32, CHA, 128)
    srcsflat = src.reshape(16, 2 * CHA * 128)
    dstsflat = dst.reshape(16, 2 * CHA * 128)

    x_pad = jnp.zeros((NPAD, HID), jnp.float32).at[:N].set(x)
    batch_pad = jnp.concatenate(
        [batch.astype(jnp.int32), jnp.full((NPAD - N,), NG, jnp.int32)])

    h2, asv, adv, mas, mad = _mm_first(x_pad, W1, a1s, a1d)
    acc, s2 = _gat_edges(h2, srcs3d, dsts3d, srcsflat, dstsflat,
                         asv, adv, mas, mad)
    h2, asv, adv, mas, mad = _mm_mid(acc, s2, b1, W2, a2s, a2d)
    acc, s2 = _gat_edges(h2, srcs3d, dsts3d, srcsflat, dstsflat,
                         asv, adv, mas, mad)
    h2, asv, adv, mas, mad = _mm_mid(acc, s2, b2, W3, a3s, a3d)
    acc, s2 = _gat_edges(h2, srcs3d, dsts3d, srcsflat, dstsflat,
                         asv, adv, mas, mad)
    return _final(batch_pad, acc, s2, b3, x_pad, W0, b0, Wn, bn, W4, b4)


def kernel(x, edge_index, batch, W1, a1s, a1d, b1, W2, a2s, a2d, b2,
           W3, a3s, a3d, b3, Wn, bn, W0, b0, W4, b4):
    return _run(x, edge_index, batch, W1, a1s, a1d, b1, W2, a2s, a2d, b2,
                W3, a3s, a3d, b3, Wn, bn, W0, b0, W4, b4)


# speedup vs baseline: 20.8405x; 1.1840x over previous
import dataclasses

import jax
import jax.numpy as jnp
import numpy as np
from jax import lax
from jax.experimental import pallas as pl
from jax.experimental.pallas import tpu as pltpu
from jax.experimental.pallas import tpu_sc as plsc

N = 10000
NPAD = 10240
E = 170000
CHA = 42
EPC = 32 * CHA * 128
CK = 128
NCHK = EPC // 16 // CK
NACC = 10048

_P32 = np.concatenate([np.arange(0, 32, 2), np.arange(1, 32, 2)])
_P128 = np.concatenate([g * 32 + _P32 for g in range(4)])
_P256 = np.concatenate([_P128, 128 + _P128])
HID = 256
NG = 64
R = 1024
NBLK = NPAD // R

_HIGH = lax.Precision.HIGHEST


def _mm(x, w, dims):
    return lax.dot_general(x, w, (dims, ((), ())), precision=_HIGH,
                           preferred_element_type=jnp.float32)


def _sc_params(tc_tiling=True):
    cp = pltpu.CompilerParams()
    if "needs_layout_passes" in pltpu.CompilerParams.__dataclass_fields__:
        cp = dataclasses.replace(cp, needs_layout_passes=False)
    if (not tc_tiling
            and "use_tc_tiling_on_sc"
            in pltpu.CompilerParams.__dataclass_fields__):
        cp = dataclasses.replace(cp, use_tc_tiling_on_sc=False)
    return cp


def _mm_first_body(x_ref, w_ref, as_ref, ad_ref,
                   h2_ref, asv_ref, adv_ref, mas_ref, mad_ref):
    i = pl.program_id(0)
    h = _mm(x_ref[...], w_ref[...], ((1,), (1,)))
    h2_ref[0] = h[:, :128].astype(jnp.bfloat16)
    h2_ref[1] = h[:, 128:].astype(jnp.bfloat16)
    asv = _mm(h, as_ref[...], ((1,), (0,)))
    adv = _mm(h, ad_ref[...], ((1,), (0,)))
    asv_ref[...] = asv
    adv_ref[...] = adv

    @pl.when(i == 0)
    def _():
        mas_ref[...] = jnp.full((16,), -3e38, jnp.float32)
        mad_ref[...] = jnp.full((16,), -3e38, jnp.float32)

    mas_ref[...] = jnp.maximum(mas_ref[...], jnp.max(asv))
    mad_ref[...] = jnp.maximum(mad_ref[...], jnp.max(adv))


def _mm_first(x_pad, w, a_s, a_d):
    return pl.pallas_call(
        _mm_first_body,
        grid=(NBLK,),
        in_specs=[
            pl.BlockSpec((R, HID), lambda i: (i, 0)),
            pl.BlockSpec((HID, HID), lambda i: (0, 0)),
            pl.BlockSpec((HID,), lambda i: (0,)),
            pl.BlockSpec((HID,), lambda i: (0,)),
        ],
        out_specs=[
            pl.BlockSpec((2, R, 128), lambda i: (0, i, 0)),
            pl.BlockSpec((R,), lambda i: (i,)),
            pl.BlockSpec((R,), lambda i: (i,)),
            pl.BlockSpec((16,), lambda i: (0,)),
            pl.BlockSpec((16,), lambda i: (0,)),
        ],
        out_shape=[
            jax.ShapeDtypeStruct((2, NPAD, 128), jnp.bfloat16),
            jax.ShapeDtypeStruct((NPAD,), jnp.float32),
            jax.ShapeDtypeStruct((NPAD,), jnp.float32),
            jax.ShapeDtypeStruct((16,), jnp.float32),
            jax.ShapeDtypeStruct((16,), jnp.float32),
        ],
    )(x_pad, w, a_s, a_d)


def _mm_mid_body(acc_ref, s_ref, b_ref, w_ref, as_ref, ad_ref,
                 h2_ref, asv_ref, adv_ref, mas_ref, mad_ref):
    i = pl.program_id(0)
    s = s_ref[0] + s_ref[1]
    rcp = 1.0 / jnp.maximum(s, 1e-30)
    b = b_ref[...]
    x0 = jnp.maximum(acc_ref[0] * rcp[:, None] + b[None, :128], 0.0)
    x1 = jnp.maximum(acc_ref[1] * rcp[:, None] + b[None, 128:], 0.0)
    w = w_ref[...]
    h = _mm(x0, w[:, :128], ((1,), (1,))) + _mm(x1, w[:, 128:], ((1,), (1,)))
    h2_ref[0] = h[:, :128].astype(jnp.bfloat16)
    h2_ref[1] = h[:, 128:].astype(jnp.bfloat16)
    asv = _mm(h, as_ref[...], ((1,), (0,)))
    adv = _mm(h, ad_ref[...], ((1,), (0,)))
    asv_ref[...] = asv
    adv_ref[...] = adv

    @pl.when(i == 0)
    def _():
        mas_ref[...] = jnp.full((16,), -3e38, jnp.float32)
        mad_ref[...] = jnp.full((16,), -3e38, jnp.float32)

    mas_ref[...] = jnp.maximum(mas_ref[...], jnp.max(asv))
    mad_ref[...] = jnp.maximum(mad_ref[...], jnp.max(adv))


def _mm_mid(acc, s2, b, w, a_s, a_d):
    return pl.pallas_call(
        _mm_mid_body,
        grid=(NBLK,),
        in_specs=[
            pl.BlockSpec((2, R, 128), lambda i: (0, i, 0)),
            pl.BlockSpec((2, R), lambda i: (0, i)),
            pl.BlockSpec((HID,), lambda i: (0,)),
            pl.BlockSpec((HID, HID), lambda i: (0, 0)),
            pl.BlockSpec((HID,), lambda i: (0,)),
            pl.BlockSpec((HID,), lambda i: (0,)),
        ],
        out_specs=[
            pl.BlockSpec((2, R, 128), lambda i: (0, i, 0)),
            pl.BlockSpec((R,), lambda i: (i,)),
            pl.BlockSpec((R,), lambda i: (i,)),
            pl.BlockSpec((16,), lambda i: (0,)),
            pl.BlockSpec((16,), lambda i: (0,)),
        ],
        out_shape=[
            jax.ShapeDtypeStruct((2, NPAD, 128), jnp.bfloat16),
            jax.ShapeDtypeStruct((NPAD,), jnp.float32),
            jax.ShapeDtypeStruct((NPAD,), jnp.float32),
            jax.ShapeDtypeStruct((16,), jnp.float32),
            jax.ShapeDtypeStruct((16,), jnp.float32),
        ],
    )(acc, s2, b, w, a_s, a_d)


def _sc_soft_body(srcs_hbm, dsts_hbm, asrc_hbm, adst_hbm, mas_hbm, mad_hbm,
                  ex_hbm, sout_hbm,
                  asrc_t, adst_t, src_t, dst_t, ex_t, mas_t, mad_t, z_t, s_sh):
    c = lax.axis_index("c")
    sid = lax.axis_index("s")
    w = 2 * sid + c

    pltpu.sync_copy(srcs_hbm.at[w], src_t)
    pltpu.sync_copy(dsts_hbm.at[w], dst_t)
    pltpu.sync_copy(asrc_hbm, asrc_t)
    pltpu.sync_copy(adst_hbm, adst_t)
    pltpu.sync_copy(mas_hbm, mas_t)
    pltpu.sync_copy(mad_hbm, mad_t)

    zv = jnp.zeros((16,), jnp.float32)

    @pl.loop(0, 640, step=16)
    def _(i):
        z_t[pl.ds(i, 16)] = zv

    pltpu.sync_copy(z_t, s_sh.at[pl.ds(sid * 640, 640)])
    plsc.subcore_barrier()

    t_m = mas_t[...] + mad_t[...]
    m_vec = jnp.maximum(t_m, 0.2 * t_m)

    @pl.loop(0, CHA)
    def _(j):
        for k in range(8):
            s16 = src_t[j, pl.ds(k * 16, 16)]
            d16 = dst_t[j, pl.ds(k * 16, 16)]
            av = plsc.load_gather(asrc_t, [s16])
            bv = plsc.load_gather(adst_t, [d16])
            t = av + bv
            e = jnp.maximum(t, 0.2 * t)
            ex_t[j, pl.ds(k * 16, 16)] = jnp.exp(e - m_vec)
        pltpu.sync_copy(ex_t.at[j], s_sh.at[dst_t.at[j]], add=True)

    pltpu.sync_copy(ex_t, ex_hbm.at[w])
    plsc.subcore_barrier()
    pltpu.sync_copy(s_sh.at[pl.ds(sid * 640, 640)],
                    sout_hbm.at[c].at[pl.ds(sid * 640, 640)])


def _sc_soft(srcs3d, dsts3d, asv, adv, mas, mad):
    mesh = plsc.VectorSubcoreMesh(core_axis_name="c", subcore_axis_name="s")
    f = pl.kernel(
        _sc_soft_body,
        compiler_params=_sc_params(),
        out_type=[
            jax.ShapeDtypeStruct((32, CHA, 128), jnp.float32),
            jax.ShapeDtypeStruct((2, NPAD), jnp.float32),
        ],
        mesh=mesh,
        scratch_types=[
            pltpu.VMEM((NPAD,), jnp.float32),
            pltpu.VMEM((NPAD,), jnp.float32),
            pltpu.VMEM((CHA, 128), jnp.int32),
            pltpu.VMEM((CHA, 128), jnp.int32),
            pltpu.VMEM((CHA, 128), jnp.float32),
            pltpu.VMEM((16,), jnp.float32),
            pltpu.VMEM((16,), jnp.float32),
            pltpu.VMEM((640,), jnp.float32),
            pltpu.VMEM_SHARED((NPAD,), jnp.float32),
        ],
    )
    return f(srcs3d, dsts3d, asv, adv, mas, mad)


def _sc_agg_body(h2_hbm, srcs_hbm, dsts_hbm, ex_hbm, acc_hbm,
                 s0_t, s1_t, d0_t, d1_t, ex0_t, ex1_t,
                 rb0_t, rb1_t, rf0_t, rf1_t,
                 semg0, semg1, seme0, seme1,
                 semi0, semi1, semd0, semd1,
                 sems0, sems1, acc_sh):
    s_t = (s0_t, s1_t)
    d_t = (d0_t, d1_t)
    exj_t = (ex0_t, ex1_t)
    rb_t = (rb0_t, rb1_t)
    rf_t = (rf0_t, rf1_t)
    semg = (semg0, semg1)
    seme = (seme0, seme1)
    semi = (semi0, semi1)
    semd = (semd0, semd1)
    sems = (sems0, sems1)
    c = lax.axis_index("c")
    sid = lax.axis_index("s")

    srcv = srcs_hbm.at[sid]
    dstv = dsts_hbm.at[sid]
    exv = ex_hbm.at[sid]
    hsel = h2_hbm.at[c]
    last = NCHK - 1

    zv = jnp.zeros((16,), jnp.float32)

    @pl.loop(0, CK)
    def _(r):
        for k in range(8):
            rf0_t[r, pl.ds(k * 16, 16)] = zv

    @pl.when(sid < 15)
    def _():
        for k in range(4):
            pltpu.sync_copy(rf0_t,
                            acc_sh.at[pl.ds(sid * 632 + k * CK, CK)])
        pltpu.sync_copy(rf0_t.at[pl.ds(0, 120)],
                        acc_sh.at[pl.ds(sid * 632 + 512, 120)])

    @pl.when(sid == 15)
    def _():
        for k in range(4):
            pltpu.sync_copy(rf0_t,
                            acc_sh.at[pl.ds(15 * 632 + k * CK, CK)])
        pltpu.sync_copy(rf0_t.at[pl.ds(0, 56)],
                        acc_sh.at[pl.ds(15 * 632 + 512, 56)])

    plsc.subcore_barrier()

    pltpu.sync_copy(srcv.at[pl.ds(0, CK)], s_t[0])
    pltpu.async_copy(hsel.at[s_t[0]], rb_t[0], semg[0])
    pltpu.async_copy(srcv.at[pl.ds(CK, CK)], s_t[1], semi[1])
    pltpu.async_copy(exv.at[pl.ds(0, CK)], exj_t[0], seme[0])
    pltpu.async_copy(dstv.at[pl.ds(0, CK)], d_t[0], semd[0])

    def step(i, p, first):
        o = 1 - p
        n1 = jnp.minimum(i + 1, last)
        n2 = jnp.minimum(i + 2, last)
        pltpu.make_async_copy(hsel.at[s_t[p]], rb_t[p], semg[p]).wait()
        pltpu.async_copy(exv.at[pl.ds(n1 * CK, CK)], exj_t[o], seme[o])
        pltpu.async_copy(srcv.at[pl.ds(n2 * CK, CK)], s_t[p], semi[p])
        pltpu.make_async_copy(srcv.at[pl.ds(0, CK)], s_t[o], semi[o]).wait()
        pltpu.async_copy(hsel.at[s_t[o]], rb_t[o], semg[o])
        pltpu.make_async_copy(exv.at[pl.ds(0, CK)], exj_t[p], seme[p]).wait()

        @plsc.parallel_loop(0, CK, unroll=4)
        def _(r):
            av = plsc.load_gather(exj_t[p], [jnp.full((16,), r, jnp.int32)])
            for k in range(4):
                seg_i = rb_t[p][r, pl.ds(k * 16, 16)]
                seg = plsc.bitcast(seg_i, jnp.bfloat16)
                a, b = plsc.unpack(seg, format=plsc.PackFormat.INTERLEAVED)
                rf_t[p][r, pl.ds(k * 32, 16)] = a * av
                rf_t[p][r, pl.ds(k * 32 + 16, 16)] = b * av

        pltpu.make_async_copy(dstv.at[pl.ds(0, CK)], d_t[p], semd[p]).wait()

        if first:
            @pl.when(i > 0)
            def _():
                pltpu.make_async_copy(rf_t[o], acc_sh.at[d_t[o]],
                                      sems[o]).wait()
        else:
            pltpu.make_async_copy(rf_t[o], acc_sh.at[d_t[o]],
                                  sems[o]).wait()

        pltpu.async_copy(rf_t[p], acc_sh.at[d_t[p]], sems[p], add=True)
        pltpu.async_copy(dstv.at[pl.ds(n1 * CK, CK)], d_t[o], semd[o])

    @pl.loop(0, NCHK // 2)
    def _(o2):
        step(2 * o2, 0, True)
        step(2 * o2 + 1, 1, False)

    pltpu.make_async_copy(rf_t[1], acc_sh.at[d_t[1]], sems[1]).wait()
    pltpu.make_async_copy(hsel.at[s_t[0]], rb_t[0], semg[0]).wait()
    pltpu.make_async_copy(exv.at[pl.ds(0, CK)], exj_t[0], seme[0]).wait()
    pltpu.make_async_copy(dstv.at[pl.ds(0, CK)], d_t[0], semd[0]).wait()
    pltpu.make_async_copy(srcv.at[pl.ds(0, CK)], s_t[1], semi[1]).wait()

    plsc.subcore_barrier()

    @pl.when(sid < 15)
    def _():
        pltpu.sync_copy(acc_sh.at[pl.ds(sid * 632, 632)],
                        acc_hbm.at[c].at[pl.ds(sid * 632, 632)])

    @pl.when(sid == 15)
    def _():
        pltpu.sync_copy(acc_sh.at[pl.ds(15 * 632, 568)],
                        acc_hbm.at[c].at[pl.ds(15 * 632, 568)])
        @pl.loop(0, CK)
        def _(r):
            for k in range(8):
                rf0_t[r, pl.ds(k * 16, 16)] = zv

        pltpu.sync_copy(rf0_t, acc_hbm.at[c].at[pl.ds(NACC, CK)])
        pltpu.sync_copy(rf0_t.at[pl.ds(0, 64)],
                        acc_hbm.at[c].at[pl.ds(NACC + CK, 64)])


def _sc_agg(h2, srcs3db, dsts3db, exb):
    mesh = plsc.VectorSubcoreMesh(core_axis_name="c", subcore_axis_name="s")
    f = pl.kernel(
        _sc_agg_body,
        compiler_params=_sc_params(tc_tiling=False),
        out_type=jax.ShapeDtypeStruct((2, NPAD, 128), jnp.float32),
        mesh=mesh,
        scratch_types=(
            [pltpu.VMEM((CK,), jnp.int32)] * 2
            + [pltpu.VMEM((CK,), jnp.int32)] * 2
            + [pltpu.VMEM((CK,), jnp.float32)] * 2
            + [pltpu.VMEM((CK, 64), jnp.int32)] * 2
            + [pltpu.VMEM((CK, 128), jnp.float32)] * 2
            + [pltpu.SemaphoreType.DMA] * 10
            + [pltpu.VMEM_SHARED((NACC, 128), jnp.float32)]
        ),
    )
    return f(h2, srcs3db, dsts3db, exb)


def _gat_edges(h2, srcs3d, dsts3d, srcsflat, dstsflat, asv, adv, mas, mad):
    ex, s2 = _sc_soft(srcs3d, dsts3d, asv, adv, mas, mad)
    h2i = lax.bitcast_convert_type(
        h2.reshape(2, NPAD, 64, 2), jnp.int32)
    acc = _sc_agg(h2i, srcsflat, dstsflat, ex.reshape(16, 2 * CHA * 128))
    return acc, s2


def _final_body(batch_sm, acc_ref, s_ref, b3_ref, bv_ref, x_ref,
                w0_ref, b0_ref, wn_ref, bn_ref, w4_ref, b4_ref,
                out_ref, hg_scr, news_scr):
    i = pl.program_id(0)

    @pl.when(i == 0)
    def _():
        hg_scr[...] = jnp.zeros((NG, HID), jnp.float32)

    s = s_ref[0] + s_ref[1]
    rcp = 1.0 / jnp.maximum(s, 1e-30)
    b3 = b3_ref[...]
    h0 = jnp.maximum(acc_ref[0] * rcp[:, None] + b3[None, :128], 0.0)
    h1 = jnp.maximum(acc_ref[1] * rcp[:, None] + b3[None, 128:], 0.0)
    h3 = jnp.concatenate([h0, h1], axis=1)
    bv = bv_ref[...]

    g_lo = batch_sm[i * R]
    g_hi = jnp.minimum(batch_sm[i * R + R - 1], NG - 1)

    def seg_body(g, _):
        mask = bv == g
        mx = jnp.max(jnp.where(mask, h3, 0.0), axis=0, keepdims=True)
        hg_scr[pl.ds(g, 1), :] = jnp.maximum(hg_scr[pl.ds(g, 1), :], mx)
        return 0

    lax.fori_loop(g_lo, g_hi + 1, seg_body, 0)

    @pl.when(i == NBLK - 1)
    def _():
        def root_body(g, _):
            def bs(_, lohi):
                lo, hi = lohi
                mid = (lo + hi) // 2
                p = batch_sm[mid] < g
                return jnp.where(p, mid + 1, lo), jnp.where(p, hi, mid)

            lo, _hi = lax.fori_loop(0, 14, bs, (0, NPAD))
            news_scr[pl.ds(g, 1), :] = x_ref[pl.ds(lo, 1), :]
            return 0

        lax.fori_loop(0, NG, root_body, 0)

        hgf = jnp.maximum(
            _mm(hg_scr[...], w0_ref[...], ((1,), (1,))) + b0_ref[...][None, :],
            0.0)
        newsh = jnp.maximum(
            _mm(news_scr[...], wn_ref[...], ((1,), (1,))) + bn_ref[...][None, :],
            0.0)
        w4 = w4_ref[...]
        logit = (_mm(hgf, w4[:, :HID], ((1,), (1,))) +
                 _mm(newsh, w4[:, HID:], ((1,), (1,))) + b4_ref[...][None, :])
        out_ref[...] = jax.nn.sigmoid(logit)


def _final(batch_pad, acc, s2, b3, x_pad, w0, b0, wn, bn, w4, b4):
    grid_spec = pltpu.PrefetchScalarGridSpec(
        num_scalar_prefetch=1,
        grid=(NBLK,),
        in_specs=[
            pl.BlockSpec((2, R, 128), lambda i, b: (0, i, 0)),
            pl.BlockSpec((2, R), lambda i, b: (0, i)),
            pl.BlockSpec((HID,), lambda i, b: (0,)),
            pl.BlockSpec((R, 1), lambda i, b: (i, 0)),
            pl.BlockSpec((NPAD, HID), lambda i, b: (0, 0)),
            pl.BlockSpec((HID, HID), lambda i, b: (0, 0)),
            pl.BlockSpec((HID,), lambda i, b: (0,)),
            pl.BlockSpec((HID, HID), lambda i, b: (0, 0)),
            pl.BlockSpec((HID,), lambda i, b: (0,)),
            pl.BlockSpec((1, 2 * HID), lambda i, b: (0, 0)),
            pl.BlockSpec((1,), lambda i, b: (0,)),
        ],
        out_specs=pl.BlockSpec((NG, 1), lambda i, b: (0, 0)),
        scratch_shapes=[
            pltpu.VMEM((NG, HID), jnp.float32),
            pltpu.VMEM((NG, HID), jnp.float32),
        ],
    )
    return pl.pallas_call(
        _final_body,
        grid_spec=grid_spec,
        out_shape=jax.ShapeDtypeStruct((NG, 1), jnp.float32),
    )(batch_pad, acc, s2, b3, batch_pad[:, None], x_pad,
      w0, b0, wn, bn, w4, b4)


@jax.jit
def _run(x, edge_index, batch, W1, a1s, a1d, b1, W2, a2s, a2d, b2,
         W3, a3s, a3d, b3, Wn, bn, W0, b0, W4, b4):
    loops = jnp.arange(N, dtype=jnp.int32)
    ei = edge_index.astype(jnp.int32)
    src = jnp.concatenate([ei[0], loops])
    dst = jnp.concatenate([ei[1], loops])
    npad_e = EPC - E
    src = jnp.concatenate([src, jnp.zeros((npad_e,), jnp.int32)])
    dst = jnp.concatenate([dst, jnp.full((npad_e,), N, jnp.int32)])
    srcs3d = src.reshape(32, CHA, 128)
    dsts3d = dst.reshape(32, CHA, 128)
    srcsflat = src.reshape(16, 2 * CHA * 128)
    dstsflat = dst.reshape(16, 2 * CHA * 128)

    x_pad = jnp.zeros((NPAD, HID), jnp.float32).at[:N].set(x)
    batch_pad = jnp.concatenate(
        [batch.astype(jnp.int32), jnp.full((NPAD - N,), NG, jnp.int32)])

    h2, asv, adv, mas, mad = _mm_first(x_pad, W1, a1s, a1d)
    acc, s2 = _gat_edges(h2, srcs3d, dsts3d, srcsflat, dstsflat,
                         asv, adv, mas, mad)
    h2, asv, adv, mas, mad = _mm_mid(acc, s2, b1[_P256], W2[:, _P256],
                                     a2s, a2d)
    acc, s2 = _gat_edges(h2, srcs3d, dsts3d, srcsflat, dstsflat,
                         asv, adv, mas, mad)
    h2, asv, adv, mas, mad = _mm_mid(acc, s2, b2[_P256], W3[:, _P256],
                                     a3s, a3d)
    acc, s2 = _gat_edges(h2, srcs3d, dsts3d, srcsflat, dstsflat,
                         asv, adv, mas, mad)
    return _final(batch_pad, acc, s2, b3[_P256], x_pad, W0[:, _P256], b0,
                  Wn, bn, W4, b4)


def kernel(x, edge_index, batch, W1, a1s, a1d, b1, W2, a2s, a2d, b2,
           W3, a3s, a3d, b3, Wn, bn, W0, b0, W4, b4):
    return _run(x, edge_index, batch, W1, a1s, a1d, b1, W2, a2s, a2d, b2,
                W3, a3s, a3d, b3, Wn, bn, W0, b0, W4, b4)


# speedup vs baseline: 20.8413x; 1.0000x over previous
import dataclasses

import jax
import jax.numpy as jnp
import numpy as np
from jax import lax
from jax.experimental import pallas as pl
from jax.experimental.pallas import tpu as pltpu
from jax.experimental.pallas import tpu_sc as plsc

N = 10000
NPAD = 10240
E = 170000
CHA = 42
EPC = 32 * CHA * 128
CK = 128
NCHK = EPC // 16 // CK
NACC = 10048

_P32 = np.concatenate([np.arange(0, 32, 2), np.arange(1, 32, 2)])
_P128 = np.concatenate([g * 32 + _P32 for g in range(4)])
_P256 = np.concatenate([_P128, 128 + _P128])
HID = 256
NG = 64
R = 1024
NBLK = NPAD // R

_HIGH = lax.Precision.HIGHEST


def _mm(x, w, dims):
    return lax.dot_general(x, w, (dims, ((), ())), precision=_HIGH,
                           preferred_element_type=jnp.float32)


def _sc_params(tc_tiling=True):
    cp = pltpu.CompilerParams()
    if "needs_layout_passes" in pltpu.CompilerParams.__dataclass_fields__:
        cp = dataclasses.replace(cp, needs_layout_passes=False)
    if (not tc_tiling
            and "use_tc_tiling_on_sc"
            in pltpu.CompilerParams.__dataclass_fields__):
        cp = dataclasses.replace(cp, use_tc_tiling_on_sc=False)
    return cp


def _mm_first_body(x_ref, w_ref, as_ref, ad_ref,
                   h2_ref, asv_ref, adv_ref, mas_ref, mad_ref):
    i = pl.program_id(0)
    h = _mm(x_ref[...], w_ref[...], ((1,), (1,)))
    h2_ref[0] = h[:, :128].astype(jnp.bfloat16)
    h2_ref[1] = h[:, 128:].astype(jnp.bfloat16)
    asv = _mm(h, as_ref[...], ((1,), (0,)))
    adv = _mm(h, ad_ref[...], ((1,), (0,)))
    asv_ref[...] = asv
    adv_ref[...] = adv

    @pl.when(i == 0)
    def _():
        mas_ref[...] = jnp.full((16,), -3e38, jnp.float32)
        mad_ref[...] = jnp.full((16,), -3e38, jnp.float32)

    mas_ref[...] = jnp.maximum(mas_ref[...], jnp.max(asv))
    mad_ref[...] = jnp.maximum(mad_ref[...], jnp.max(adv))


def _mm_first(x_pad, w, a_s, a_d):
    return pl.pallas_call(
        _mm_first_body,
        grid=(NBLK,),
        in_specs=[
            pl.BlockSpec((R, HID), lambda i: (i, 0)),
            pl.BlockSpec((HID, HID), lambda i: (0, 0)),
            pl.BlockSpec((HID,), lambda i: (0,)),
            pl.BlockSpec((HID,), lambda i: (0,)),
        ],
        out_specs=[
            pl.BlockSpec((2, R, 128), lambda i: (0, i, 0)),
            pl.BlockSpec((R,), lambda i: (i,)),
            pl.BlockSpec((R,), lambda i: (i,)),
            pl.BlockSpec((16,), lambda i: (0,)),
            pl.BlockSpec((16,), lambda i: (0,)),
        ],
        out_shape=[
            jax.ShapeDtypeStruct((2, NPAD, 128), jnp.bfloat16),
            jax.ShapeDtypeStruct((NPAD,), jnp.float32),
            jax.ShapeDtypeStruct((NPAD,), jnp.float32),
            jax.ShapeDtypeStruct((16,), jnp.float32),
            jax.ShapeDtypeStruct((16,), jnp.float32),
        ],
    )(x_pad, w, a_s, a_d)


def _mm_mid_body(acc_ref, s_ref, b_ref, w_ref, as_ref, ad_ref,
                 h2_ref, asv_ref, adv_ref, mas_ref, mad_ref):
    i = pl.program_id(0)
    s = s_ref[0] + s_ref[1]
    rcp = 1.0 / jnp.maximum(s, 1e-30)
    b = b_ref[...]
    x0 = jnp.maximum(acc_ref[0] * rcp[:, None] + b[None, :128], 0.0)
    x1 = jnp.maximum(acc_ref[1] * rcp[:, None] + b[None, 128:], 0.0)
    w = w_ref[...]
    h = _mm(x0, w[:, :128], ((1,), (1,))) + _mm(x1, w[:, 128:], ((1,), (1,)))
    h2_ref[0] = h[:, :128].astype(jnp.bfloat16)
    h2_ref[1] = h[:, 128:].astype(jnp.bfloat16)
    asv = _mm(h, as_ref[...], ((1,), (0,)))
    adv = _mm(h, ad_ref[...], ((1,), (0,)))
    asv_ref[...] = asv
    adv_ref[...] = adv

    @pl.when(i == 0)
    def _():
        mas_ref[...] = jnp.full((16,), -3e38, jnp.float32)
        mad_ref[...] = jnp.full((16,), -3e38, jnp.float32)

    mas_ref[...] = jnp.maximum(mas_ref[...], jnp.max(asv))
    mad_ref[...] = jnp.maximum(mad_ref[...], jnp.max(adv))


def _mm_mid(acc, s2, b, w, a_s, a_d):
    return pl.pallas_call(
        _mm_mid_body,
        grid=(NBLK,),
        in_specs=[
            pl.BlockSpec((2, R, 128), lambda i: (0, i, 0)),
            pl.BlockSpec((2, R), lambda i: (0, i)),
            pl.BlockSpec((HID,), lambda i: (0,)),
            pl.BlockSpec((HID, HID), lambda i: (0, 0)),
            pl.BlockSpec((HID,), lambda i: (0,)),
            pl.BlockSpec((HID,), lambda i: (0,)),
        ],
        out_specs=[
            pl.BlockSpec((2, R, 128), lambda i: (0, i, 0)),
            pl.BlockSpec((R,), lambda i: (i,)),
            pl.BlockSpec((R,), lambda i: (i,)),
            pl.BlockSpec((16,), lambda i: (0,)),
            pl.BlockSpec((16,), lambda i: (0,)),
        ],
        out_shape=[
            jax.ShapeDtypeStruct((2, NPAD, 128), jnp.bfloat16),
            jax.ShapeDtypeStruct((NPAD,), jnp.float32),
            jax.ShapeDtypeStruct((NPAD,), jnp.float32),
            jax.ShapeDtypeStruct((16,), jnp.float32),
            jax.ShapeDtypeStruct((16,), jnp.float32),
        ],
    )(acc, s2, b, w, a_s, a_d)


def _sc_soft_body(srcs_hbm, dsts_hbm, asrc_hbm, adst_hbm, mas_hbm, mad_hbm,
                  ex_hbm, sout_hbm,
                  asrc_t, adst_t, src_t, dst_t, ex_t, mas_t, mad_t, z_t, s_sh):
    c = lax.axis_index("c")
    sid = lax.axis_index("s")
    w = 2 * sid + c

    pltpu.sync_copy(srcs_hbm.at[w], src_t)
    pltpu.sync_copy(dsts_hbm.at[w], dst_t)
    pltpu.sync_copy(asrc_hbm, asrc_t)
    pltpu.sync_copy(adst_hbm, adst_t)
    pltpu.sync_copy(mas_hbm, mas_t)
    pltpu.sync_copy(mad_hbm, mad_t)

    zv = jnp.zeros((16,), jnp.float32)

    @pl.loop(0, 640, step=16)
    def _(i):
        z_t[pl.ds(i, 16)] = zv

    pltpu.sync_copy(z_t, s_sh.at[pl.ds(sid * 640, 640)])
    plsc.subcore_barrier()

    t_m = mas_t[...] + mad_t[...]
    m_vec = jnp.maximum(t_m, 0.2 * t_m)

    @pl.loop(0, CHA)
    def _(j):
        for k in range(8):
            s16 = src_t[j, pl.ds(k * 16, 16)]
            d16 = dst_t[j, pl.ds(k * 16, 16)]
            av = plsc.load_gather(asrc_t, [s16])
            bv = plsc.load_gather(adst_t, [d16])
            t = av + bv
            e = jnp.maximum(t, 0.2 * t)
            ex_t[j, pl.ds(k * 16, 16)] = jnp.exp(e - m_vec)
        pltpu.sync_copy(ex_t.at[j], s_sh.at[dst_t.at[j]], add=True)

    pltpu.sync_copy(ex_t, ex_hbm.at[w])
    plsc.subcore_barrier()
    pltpu.sync_copy(s_sh.at[pl.ds(sid * 640, 640)],
                    sout_hbm.at[c].at[pl.ds(sid * 640, 640)])


def _sc_soft(srcs3d, dsts3d, asv, adv, mas, mad):
    mesh = plsc.VectorSubcoreMesh(core_axis_name="c", subcore_axis_name="s")
    f = pl.kernel(
        _sc_soft_body,
        compiler_params=_sc_params(),
        out_type=[
            jax.ShapeDtypeStruct((32, CHA, 128), jnp.float32),
            jax.ShapeDtypeStruct((2, NPAD), jnp.float32),
        ],
        mesh=mesh,
        scratch_types=[
            pltpu.VMEM((NPAD,), jnp.float32),
            pltpu.VMEM((NPAD,), jnp.float32),
            pltpu.VMEM((CHA, 128), jnp.int32),
            pltpu.VMEM((CHA, 128), jnp.int32),
            pltpu.VMEM((CHA, 128), jnp.float32),
            pltpu.VMEM((16,), jnp.float32),
            pltpu.VMEM((16,), jnp.float32),
            pltpu.VMEM((640,), jnp.float32),
            pltpu.VMEM_SHARED((NPAD,), jnp.float32),
        ],
    )
    return f(srcs3d, dsts3d, asv, adv, mas, mad)


def _sc_agg_body(h2_hbm, srcs_hbm, dsts_hbm, ex_hbm, acc_hbm,
                 s0_t, s1_t, d0_t, d1_t, ex0_t, ex1_t,
                 rb0_t, rb1_t, rf0_t, rf1_t,
                 semg0, semg1, seme0, seme1,
                 semi0, semi1, semd0, semd1,
                 sems0, sems1, acc_sh):
    s_t = (s0_t, s1_t)
    d_t = (d0_t, d1_t)
    exj_t = (ex0_t, ex1_t)
    rb_t = (rb0_t, rb1_t)
    rf_t = (rf0_t, rf1_t)
    semg = (semg0, semg1)
    seme = (seme0, seme1)
    semi = (semi0, semi1)
    semd = (semd0, semd1)
    sems = (sems0, sems1)
    c = lax.axis_index("c")
    sid = lax.axis_index("s")

    srcv = srcs_hbm.at[sid]
    dstv = dsts_hbm.at[sid]
    exv = ex_hbm.at[sid]
    hsel = h2_hbm.at[c]
    last = NCHK - 1

    zv = jnp.zeros((16,), jnp.float32)

    @pl.loop(0, CK)
    def _(r):
        for k in range(8):
            rf0_t[r, pl.ds(k * 16, 16)] = zv

    @pl.when(sid < 15)
    def _():
        for k in range(4):
            pltpu.sync_copy(rf0_t,
                            acc_sh.at[pl.ds(sid * 632 + k * CK, CK)])
        pltpu.sync_copy(rf0_t.at[pl.ds(0, 120)],
                        acc_sh.at[pl.ds(sid * 632 + 512, 120)])

    @pl.when(sid == 15)
    def _():
        for k in range(4):
            pltpu.sync_copy(rf0_t,
                            acc_sh.at[pl.ds(15 * 632 + k * CK, CK)])
        pltpu.sync_copy(rf0_t.at[pl.ds(0, 56)],
                        acc_sh.at[pl.ds(15 * 632 + 512, 56)])

    plsc.subcore_barrier()

    pltpu.sync_copy(srcv.at[pl.ds(0, CK)], s_t[0])
    pltpu.async_copy(hsel.at[s_t[0]], rb_t[0], semg[0])
    pltpu.async_copy(srcv.at[pl.ds(CK, CK)], s_t[1], semi[1])
    pltpu.async_copy(exv.at[pl.ds(0, CK)], exj_t[0], seme[0])
    pltpu.async_copy(dstv.at[pl.ds(0, CK)], d_t[0], semd[0])

    def step(i, p, first):
        o = 1 - p
        n1 = jnp.minimum(i + 1, last)
        n2 = jnp.minimum(i + 2, last)
        pltpu.make_async_copy(hsel.at[s_t[p]], rb_t[p], semg[p]).wait()
        pltpu.async_copy(exv.at[pl.ds(n1 * CK, CK)], exj_t[o], seme[o])
        pltpu.async_copy(srcv.at[pl.ds(n2 * CK, CK)], s_t[p], semi[p])
        pltpu.make_async_copy(srcv.at[pl.ds(0, CK)], s_t[o], semi[o]).wait()
        pltpu.async_copy(hsel.at[s_t[o]], rb_t[o], semg[o])
        pltpu.make_async_copy(exv.at[pl.ds(0, CK)], exj_t[p], seme[p]).wait()

        @plsc.parallel_loop(0, CK, unroll=8)
        def _(r):
            av = plsc.load_gather(exj_t[p], [jnp.full((16,), r, jnp.int32)])
            for k in range(4):
                seg_i = rb_t[p][r, pl.ds(k * 16, 16)]
                seg = plsc.bitcast(seg_i, jnp.bfloat16)
                a, b = plsc.unpack(seg, format=plsc.PackFormat.INTERLEAVED)
                rf_t[p][r, pl.ds(k * 32, 16)] = a * av
                rf_t[p][r, pl.ds(k * 32 + 16, 16)] = b * av

        pltpu.make_async_copy(dstv.at[pl.ds(0, CK)], d_t[p], semd[p]).wait()

        if first:
            @pl.when(i > 0)
            def _():
                pltpu.make_async_copy(rf_t[o], acc_sh.at[d_t[o]],
                                      sems[o]).wait()
        else:
            pltpu.make_async_copy(rf_t[o], acc_sh.at[d_t[o]],
                                  sems[o]).wait()

        pltpu.async_copy(rf_t[p], acc_sh.at[d_t[p]], sems[p], add=True)
        pltpu.async_copy(dstv.at[pl.ds(n1 * CK, CK)], d_t[o], semd[o])

    @pl.loop(0, NCHK // 2)
    def _(o2):
        step(2 * o2, 0, True)
        step(2 * o2 + 1, 1, False)

    pltpu.make_async_copy(rf_t[1], acc_sh.at[d_t[1]], sems[1]).wait()
    pltpu.make_async_copy(hsel.at[s_t[0]], rb_t[0], semg[0]).wait()
    pltpu.make_async_copy(exv.at[pl.ds(0, CK)], exj_t[0], seme[0]).wait()
    pltpu.make_async_copy(dstv.at[pl.ds(0, CK)], d_t[0], semd[0]).wait()
    pltpu.make_async_copy(srcv.at[pl.ds(0, CK)], s_t[1], semi[1]).wait()

    plsc.subcore_barrier()

    @pl.when(sid < 15)
    def _():
        pltpu.sync_copy(acc_sh.at[pl.ds(sid * 632, 632)],
                        acc_hbm.at[c].at[pl.ds(sid * 632, 632)])

    @pl.when(sid == 15)
    def _():
        pltpu.sync_copy(acc_sh.at[pl.ds(15 * 632, 568)],
                        acc_hbm.at[c].at[pl.ds(15 * 632, 568)])
        @pl.loop(0, CK)
        def _(r):
            for k in range(8):
                rf0_t[r, pl.ds(k * 16, 16)] = zv

        pltpu.sync_copy(rf0_t, acc_hbm.at[c].at[pl.ds(NACC, CK)])
        pltpu.sync_copy(rf0_t.at[pl.ds(0, 64)],
                        acc_hbm.at[c].at[pl.ds(NACC + CK, 64)])


def _sc_agg(h2, srcs3db, dsts3db, exb):
    mesh = plsc.VectorSubcoreMesh(core_axis_name="c", subcore_axis_name="s")
    f = pl.kernel(
        _sc_agg_body,
        compiler_params=_sc_params(tc_tiling=False),
        out_type=jax.ShapeDtypeStruct((2, NPAD, 128), jnp.float32),
        mesh=mesh,
        scratch_types=(
            [pltpu.VMEM((CK,), jnp.int32)] * 2
            + [pltpu.VMEM((CK,), jnp.int32)] * 2
            + [pltpu.VMEM((CK,), jnp.float32)] * 2
            + [pltpu.VMEM((CK, 64), jnp.int32)] * 2
            + [pltpu.VMEM((CK, 128), jnp.float32)] * 2
            + [pltpu.SemaphoreType.DMA] * 10
            + [pltpu.VMEM_SHARED((NACC, 128), jnp.float32)]
        ),
    )
    return f(h2, srcs3db, dsts3db, exb)


def _gat_edges(h2, srcs3d, dsts3d, srcsflat, dstsflat, asv, adv, mas, mad):
    ex, s2 = _sc_soft(srcs3d, dsts3d, asv, adv, mas, mad)
    h2i = lax.bitcast_convert_type(
        h2.reshape(2, NPAD, 64, 2), jnp.int32)
    acc = _sc_agg(h2i, srcsflat, dstsflat, ex.reshape(16, 2 * CHA * 128))
    return acc, s2


def _final_body(batch_sm, acc_ref, s_ref, b3_ref, bv_ref, x_ref,
                w0_ref, b0_ref, wn_ref, bn_ref, w4_ref, b4_ref,
                out_ref, hg_scr, news_scr):
    i = pl.program_id(0)

    @pl.when(i == 0)
    def _():
        hg_scr[...] = jnp.zeros((NG, HID), jnp.float32)

    s = s_ref[0] + s_ref[1]
    rcp = 1.0 / jnp.maximum(s, 1e-30)
    b3 = b3_ref[...]
    h0 = jnp.maximum(acc_ref[0] * rcp[:, None] + b3[None, :128], 0.0)
    h1 = jnp.maximum(acc_ref[1] * rcp[:, None] + b3[None, 128:], 0.0)
    h3 = jnp.concatenate([h0, h1], axis=1)
    bv = bv_ref[...]

    g_lo = batch_sm[i * R]
    g_hi = jnp.minimum(batch_sm[i * R + R - 1], NG - 1)

    def seg_body(g, _):
        mask = bv == g
        mx = jnp.max(jnp.where(mask, h3, 0.0), axis=0, keepdims=True)
        hg_scr[pl.ds(g, 1), :] = jnp.maximum(hg_scr[pl.ds(g, 1), :], mx)
        return 0

    lax.fori_loop(g_lo, g_hi + 1, seg_body, 0)

    @pl.when(i == NBLK - 1)
    def _():
        def root_body(g, _):
            def bs(_, lohi):
                lo, hi = lohi
                mid = (lo + hi) // 2
                p = batch_sm[mid] < g
                return jnp.where(p, mid + 1, lo), jnp.where(p, hi, mid)

            lo, _hi = lax.fori_loop(0, 14, bs, (0, NPAD))
            news_scr[pl.ds(g, 1), :] = x_ref[pl.ds(lo, 1), :]
            return 0

        lax.fori_loop(0, NG, root_body, 0)

        hgf = jnp.maximum(
            _mm(hg_scr[...], w0_ref[...], ((1,), (1,))) + b0_ref[...][None, :],
            0.0)
        newsh = jnp.maximum(
            _mm(news_scr[...], wn_ref[...], ((1,), (1,))) + bn_ref[...][None, :],
            0.0)
        w4 = w4_ref[...]
        logit = (_mm(hgf, w4[:, :HID], ((1,), (1,))) +
                 _mm(newsh, w4[:, HID:], ((1,), (1,))) + b4_ref[...][None, :])
        out_ref[...] = jax.nn.sigmoid(logit)


def _final(batch_pad, acc, s2, b3, x_pad, w0, b0, wn, bn, w4, b4):
    grid_spec = pltpu.PrefetchScalarGridSpec(
        num_scalar_prefetch=1,
        grid=(NBLK,),
        in_specs=[
            pl.BlockSpec((2, R, 128), lambda i, b: (0, i, 0)),
            pl.BlockSpec((2, R), lambda i, b: (0, i)),
            pl.BlockSpec((HID,), lambda i, b: (0,)),
            pl.BlockSpec((R, 1), lambda i, b: (i, 0)),
            pl.BlockSpec((NPAD, HID), lambda i, b: (0, 0)),
            pl.BlockSpec((HID, HID), lambda i, b: (0, 0)),
            pl.BlockSpec((HID,), lambda i, b: (0,)),
            pl.BlockSpec((HID, HID), lambda i, b: (0, 0)),
            pl.BlockSpec((HID,), lambda i, b: (0,)),
            pl.BlockSpec((1, 2 * HID), lambda i, b: (0, 0)),
            pl.BlockSpec((1,), lambda i, b: (0,)),
        ],
        out_specs=pl.BlockSpec((NG, 1), lambda i, b: (0, 0)),
        scratch_shapes=[
            pltpu.VMEM((NG, HID), jnp.float32),
            pltpu.VMEM((NG, HID), jnp.float32),
        ],
    )
    return pl.pallas_call(
        _final_body,
        grid_spec=grid_spec,
        out_shape=jax.ShapeDtypeStruct((NG, 1), jnp.float32),
    )(batch_pad, acc, s2, b3, batch_pad[:, None], x_pad,
      w0, b0, wn, bn, w4, b4)


@jax.jit
def _run(x, edge_index, batch, W1, a1s, a1d, b1, W2, a2s, a2d, b2,
         W3, a3s, a3d, b3, Wn, bn, W0, b0, W4, b4):
    loops = jnp.arange(N, dtype=jnp.int32)
    ei = edge_index.astype(jnp.int32)
    src = jnp.concatenate([ei[0], loops])
    dst = jnp.concatenate([ei[1], loops])
    npad_e = EPC - E
    src = jnp.concatenate([src, jnp.zeros((npad_e,), jnp.int32)])
    dst = jnp.concatenate([dst, jnp.full((npad_e,), N, jnp.int32)])
    srcs3d = src.reshape(32, CHA, 128)
    dsts3d = dst.reshape(32, CHA, 128)
    srcsflat = src.reshape(16, 2 * CHA * 128)
    dstsflat = dst.reshape(16, 2 * CHA * 128)

    x_pad = jnp.zeros((NPAD, HID), jnp.float32).at[:N].set(x)
    batch_pad = jnp.concatenate(
        [batch.astype(jnp.int32), jnp.full((NPAD - N,), NG, jnp.int32)])

    h2, asv, adv, mas, mad = _mm_first(x_pad, W1, a1s, a1d)
    acc, s2 = _gat_edges(h2, srcs3d, dsts3d, srcsflat, dstsflat,
                         asv, adv, mas, mad)
    h2, asv, adv, mas, mad = _mm_mid(acc, s2, b1[_P256], W2[:, _P256],
                                     a2s, a2d)
    acc, s2 = _gat_edges(h2, srcs3d, dsts3d, srcsflat, dstsflat,
                         asv, adv, mas, mad)
    h2, asv, adv, mas, mad = _mm_mid(acc, s2, b2[_P256], W3[:, _P256],
                                     a3s, a3d)
    acc, s2 = _gat_edges(h2, srcs3d, dsts3d, srcsflat, dstsflat,
                         asv, adv, mas, mad)
    return _final(batch_pad, acc, s2, b3[_P256], x_pad, W0[:, _P256], b0,
                  Wn, bn, W4, b4)


def kernel(x, edge_index, batch, W1, a1s, a1d, b1, W2, a2s, a2d, b2,
           W3, a3s, a3d, b3, Wn, bn, W0, b0, W4, b4):
    return _run(x, edge_index, batch, W1, a1s, a1d, b1, W2, a2s, a2d, b2,
                W3, a3s, a3d, b3, Wn, bn, W0, b0, W4, b4)
